# Initial kernel scaffold; baseline (speedup 1.0000x reference)
import jax
import jax.numpy as jnp
from jax.experimental import pallas as pl


def kernel(f, route, mask, edge, node, A, W_node, emb_edges, W_ev, We_U, be_U, We_V, be_V, Wn_U, bn_U, Wn_V, bn_V, g_e, b_e, g_n, b_n, W_el, b_el, emb_time, Wq, bq, Wk, bk, Wv, bv, Wd, bd):
    raise NotImplementedError("write your pallas kernel here")


# speedup vs baseline: 1.9974x; 1.9974x over previous
import math

import jax
import jax.numpy as jnp
import numpy as np
from jax.experimental import pallas as pl
from jax.experimental.pallas import tpu as pltpu

B = 16; T = 24; N = 64; SEQ = 25
HID = 128; NODE_DIM = 8; VOC = 3; EVD = 5; TED = 16; EOD = 24; NL = 2
ATT = 64; H = 8; DH = ATT // H
NN = N * N
EPS = 1e-5


def _pe_const():
    pe = []
    for pos in range(SEQ):
        row = []
        for ii in range(0, EOD, 2):
            row.append(math.sin(pos / 10000 ** (2 * ii / EOD)))
            row.append(math.cos(pos / 10000 ** (2 * ii / EOD)))
        pe.append(row)
    return jnp.asarray(np.array(pe, dtype=np.float32))


def _ka_body(ti_ref, edge_ref, node_ref, A_ref, emb_edges_ref, W_ev_ref,
             W_node_ref, WeU_ref, beU_ref, WeV_ref, beV_ref, WnU_ref,
             bnU_ref, WnV_ref, bnV_ref,
             e0_ref, etmp0_ref, x0_ref, xtmp0_ref, se_ref, sse_ref,
             sx_ref, ssx_ref):
    edge_b = edge_ref[0]
    ev = edge_b[:, :, 2:].reshape(NN, EVD)
    e_vals = jnp.dot(ev, W_ev_ref[...], preferred_element_type=jnp.float32)
    A_b = A_ref[0]
    emb = emb_edges_ref[...]
    e_tags = ((A_b == 0).astype(jnp.float32) * emb[0:1, :]
              + (A_b == 1).astype(jnp.float32) * emb[1:2, :]
              + (A_b == 2).astype(jnp.float32) * emb[2:3, :])
    e0 = jnp.concatenate([e_vals, e_tags], axis=1)
    e0_ref[0] = e0
    x0 = jnp.dot(node_ref[0][:, 1:], W_node_ref[...],
                 preferred_element_type=jnp.float32)
    x0_ref[0] = x0

    Ue = jnp.dot(e0, WeU_ref[...], preferred_element_type=jnp.float32) \
        + beU_ref[...]
    Vx = jnp.dot(x0, WeV_ref[...], preferred_element_type=jnp.float32) \
        + beV_ref[...]
    Ue3 = Ue.reshape(N, N, HID)
    e_tmp = Ue3 + Vx[None, :, :] + Vx[:, None, :]
    e_tmp_f = e_tmp.reshape(NN, HID)
    etmp0_ref[0] = e_tmp_f
    se_ref[0] = jnp.sum(e_tmp_f, axis=0, keepdims=True)
    sse_ref[0] = jnp.sum(e_tmp_f * e_tmp_f, axis=0, keepdims=True)

    gate = jax.nn.sigmoid(e_tmp)
    Vx2 = jnp.dot(x0, WnV_ref[...], preferred_element_type=jnp.float32) \
        + bnV_ref[...]
    num = jnp.sum(gate * Vx2[None, :, :], axis=1)
    den = jnp.sum(gate, axis=1)
    Ux = jnp.dot(x0, WnU_ref[...], preferred_element_type=jnp.float32) \
        + bnU_ref[...]
    x_tmp = Ux + num / (1e-20 + den)
    xtmp0_ref[0] = x_tmp
    sx_ref[0] = jnp.sum(x_tmp, axis=0, keepdims=True)
    ssx_ref[0] = jnp.sum(x_tmp * x_tmp, axis=0, keepdims=True)


def _run_ka(ti, edge, node, A, emb_edges, W_ev, W_node, WeU, beU, WeV, beV,
            WnU, bnU, WnV, bnV):
    f32 = jnp.float32
    grid_spec = pltpu.PrefetchScalarGridSpec(
        num_scalar_prefetch=1,
        grid=(B,),
        in_specs=[
            pl.BlockSpec((1, N, N, EVD + 2), lambda b, ti: (ti[b], 0, 0, 0)),
            pl.BlockSpec((1, N, NODE_DIM + 1), lambda b, ti: (ti[b], 0, 0)),
            pl.BlockSpec((1, NN, 1), lambda b, ti: (ti[b], 0, 0)),
            pl.BlockSpec((VOC, HID // 2), lambda b, ti: (0, 0)),
            pl.BlockSpec((EVD, HID // 2), lambda b, ti: (0, 0)),
            pl.BlockSpec((NODE_DIM, HID), lambda b, ti: (0, 0)),
            pl.BlockSpec((HID, HID), lambda b, ti: (0, 0)),
            pl.BlockSpec((1, HID), lambda b, ti: (0, 0)),
            pl.BlockSpec((HID, HID), lambda b, ti: (0, 0)),
            pl.BlockSpec((1, HID), lambda b, ti: (0, 0)),
            pl.BlockSpec((HID, HID), lambda b, ti: (0, 0)),
            pl.BlockSpec((1, HID), lambda b, ti: (0, 0)),
            pl.BlockSpec((HID, HID), lambda b, ti: (0, 0)),
            pl.BlockSpec((1, HID), lambda b, ti: (0, 0)),
        ],
        out_specs=[
            pl.BlockSpec((1, NN, HID), lambda b, ti: (b, 0, 0)),
            pl.BlockSpec((1, NN, HID), lambda b, ti: (b, 0, 0)),
            pl.BlockSpec((1, N, HID), lambda b, ti: (b, 0, 0)),
            pl.BlockSpec((1, N, HID), lambda b, ti: (b, 0, 0)),
            pl.BlockSpec((1, 1, HID), lambda b, ti: (b, 0, 0)),
            pl.BlockSpec((1, 1, HID), lambda b, ti: (b, 0, 0)),
            pl.BlockSpec((1, 1, HID), lambda b, ti: (b, 0, 0)),
            pl.BlockSpec((1, 1, HID), lambda b, ti: (b, 0, 0)),
        ],
    )
    out_shape = [
        jax.ShapeDtypeStruct((B, NN, HID), f32),
        jax.ShapeDtypeStruct((B, NN, HID), f32),
        jax.ShapeDtypeStruct((B, N, HID), f32),
        jax.ShapeDtypeStruct((B, N, HID), f32),
        jax.ShapeDtypeStruct((B, 1, HID), f32),
        jax.ShapeDtypeStruct((B, 1, HID), f32),
        jax.ShapeDtypeStruct((B, 1, HID), f32),
        jax.ShapeDtypeStruct((B, 1, HID), f32),
    ]
    return pl.pallas_call(_ka_body, grid_spec=grid_spec,
                          out_shape=out_shape)(
        ti, edge, node, A, emb_edges, W_ev, W_node,
        WeU, beU.reshape(1, HID), WeV, beV.reshape(1, HID),
        WnU, bnU.reshape(1, HID), WnV, bnV.reshape(1, HID))


def _kb_body(e0_ref, etmp0_ref, x0_ref, xtmp0_ref,
             se_ref, sse_ref, sx_ref, ssx_ref,
             ge_ref, be_ref, gn_ref, bn_ref,
             WeU_ref, beU_ref, WeV_ref, beV_ref,
             e1_ref, etmp1_ref, se1_ref, sse1_ref):
    cnt_e = float(B * NN)
    cnt_x = float(B * N)
    m_e = jnp.sum(se_ref[...], axis=0) / cnt_e
    v_e = jnp.sum(sse_ref[...], axis=0) / cnt_e - m_e * m_e
    m_x = jnp.sum(sx_ref[...], axis=0) / cnt_x
    v_x = jnp.sum(ssx_ref[...], axis=0) / cnt_x - m_x * m_x

    x_tmp = xtmp0_ref[0]
    xn = gn_ref[...] * (x_tmp - m_x) * jax.lax.rsqrt(v_x + EPS) + bn_ref[...]
    x1 = x0_ref[0] + jnp.maximum(xn, 0.0)

    e_tmp = etmp0_ref[0]
    en = ge_ref[...] * (e_tmp - m_e) * jax.lax.rsqrt(v_e + EPS) + be_ref[...]
    e1 = e0_ref[0] + jnp.maximum(en, 0.0)
    e1_ref[0] = e1

    Ue = jnp.dot(e1, WeU_ref[...], preferred_element_type=jnp.float32) \
        + beU_ref[...]
    Vx = jnp.dot(x1, WeV_ref[...], preferred_element_type=jnp.float32) \
        + beV_ref[...]
    e_tmp1 = Ue.reshape(N, N, HID) + Vx[None, :, :] + Vx[:, None, :]
    e_tmp1_f = e_tmp1.reshape(NN, HID)
    etmp1_ref[0] = e_tmp1_f
    se1_ref[0] = jnp.sum(e_tmp1_f, axis=0, keepdims=True)
    sse1_ref[0] = jnp.sum(e_tmp1_f * e_tmp1_f, axis=0, keepdims=True)


def _run_kb(e0, etmp0, x0, xtmp0, se, sse, sx, ssx,
            ge, be, gn, bn, WeU, beU, WeV, beV):
    f32 = jnp.float32
    full_stats = pl.BlockSpec((B, 1, HID), lambda b: (0, 0, 0))
    grid_spec = pl.GridSpec(
        grid=(B,),
        in_specs=[
            pl.BlockSpec((1, NN, HID), lambda b: (b, 0, 0)),
            pl.BlockSpec((1, NN, HID), lambda b: (b, 0, 0)),
            pl.BlockSpec((1, N, HID), lambda b: (b, 0, 0)),
            pl.BlockSpec((1, N, HID), lambda b: (b, 0, 0)),
            full_stats, full_stats, full_stats, full_stats,
            pl.BlockSpec((1, HID), lambda b: (0, 0)),
            pl.BlockSpec((1, HID), lambda b: (0, 0)),
            pl.BlockSpec((1, HID), lambda b: (0, 0)),
            pl.BlockSpec((1, HID), lambda b: (0, 0)),
            pl.BlockSpec((HID, HID), lambda b: (0, 0)),
            pl.BlockSpec((1, HID), lambda b: (0, 0)),
            pl.BlockSpec((HID, HID), lambda b: (0, 0)),
            pl.BlockSpec((1, HID), lambda b: (0, 0)),
        ],
        out_specs=[
            pl.BlockSpec((1, NN, HID), lambda b: (b, 0, 0)),
            pl.BlockSpec((1, NN, HID), lambda b: (b, 0, 0)),
            pl.BlockSpec((1, 1, HID), lambda b: (b, 0, 0)),
            pl.BlockSpec((1, 1, HID), lambda b: (b, 0, 0)),
        ],
    )
    out_shape = [
        jax.ShapeDtypeStruct((B, NN, HID), f32),
        jax.ShapeDtypeStruct((B, NN, HID), f32),
        jax.ShapeDtypeStruct((B, 1, HID), f32),
        jax.ShapeDtypeStruct((B, 1, HID), f32),
    ]
    return pl.pallas_call(_kb_body, grid_spec=grid_spec,
                          out_shape=out_shape)(
        e0, etmp0, x0, xtmp0, se, sse, sx, ssx,
        ge.reshape(1, HID), be.reshape(1, HID),
        gn.reshape(1, HID), bn.reshape(1, HID),
        WeU, beU.reshape(1, HID), WeV, beV.reshape(1, HID))


def _kc_body(tidx_ref, e1_ref, etmp1_ref, se1_ref, sse1_ref,
             ge_ref, be_ref, route_ref, mask_ref, pe_ref, emb_time_ref,
             Wel_ref, bel_ref, Wq_ref, bq_ref, Wk_ref, bk_ref,
             Wv_ref, bv_ref, Wd_ref, bd_ref, out_ref):
    cnt_e = float(B * NN)
    m_e = jnp.sum(se1_ref[...], axis=0) / cnt_e
    v_e = jnp.sum(sse1_ref[...], axis=0) / cnt_e - m_e * m_e

    e_tmp = etmp1_ref[0]
    en = ge_ref[...] * (e_tmp - m_e) * jax.lax.rsqrt(v_e + EPS) + be_ref[...]
    e2 = e1_ref[0] + jnp.maximum(en, 0.0)

    idx = route_ref[0]
    onehot = (idx == jax.lax.broadcasted_iota(
        jnp.int32, (SEQ, NN), 1)).astype(jnp.float32)
    rows = jnp.dot(onehot, e2, preferred_element_type=jnp.float32)
    r_edge = jnp.dot(rows, Wel_ref[...],
                     preferred_element_type=jnp.float32) + bel_ref[...]
    t_emb = jnp.broadcast_to(emb_time_ref[0, 0].reshape(1, TED), (SEQ, TED))
    R = jnp.concatenate([r_edge, pe_ref[...], t_emb], axis=1)

    q = jnp.dot(R, Wq_ref[...], preferred_element_type=jnp.float32) \
        + bq_ref[...]
    k = jnp.dot(R, Wk_ref[...], preferred_element_type=jnp.float32) \
        + bk_ref[...]
    v = jnp.dot(R, Wv_ref[...], preferred_element_type=jnp.float32) \
        + bv_ref[...]
    mask0 = mask_ref[0] == 0
    scale = 1.0 / math.sqrt(DH)
    ctx_heads = []
    for h in range(H):
        sl = slice(h * DH, (h + 1) * DH)
        qh = q[:, sl]
        kh = k[:, sl]
        vh = v[:, sl]
        s = jax.lax.dot_general(qh, kh, (((1,), (1,)), ((), ())),
                                preferred_element_type=jnp.float32) * scale
        s = jnp.where(mask0, 1e-8, s)
        s = s - jnp.max(s, axis=1, keepdims=True)
        p = jnp.exp(s)
        p = p / jnp.sum(p, axis=1, keepdims=True)
        ctx_heads.append(jnp.dot(p, vh, preferred_element_type=jnp.float32))
    ctx = jnp.concatenate(ctx_heads, axis=1)
    out_ref[0] = jnp.dot(ctx, Wd_ref[...],
                         preferred_element_type=jnp.float32) + bd_ref[...]


def _run_kc(tidx, e1, etmp1, se1, sse1, ge, be, route, mask, pe, emb_time,
            Wel, bel, Wq, bq, Wk, bk, Wv, bv, Wd, bd):
    f32 = jnp.float32
    full_stats = pl.BlockSpec((B, 1, HID), lambda b, t: (0, 0, 0))
    grid_spec = pltpu.PrefetchScalarGridSpec(
        num_scalar_prefetch=1,
        grid=(B,),
        in_specs=[
            pl.BlockSpec((1, NN, HID), lambda b, t: (b, 0, 0)),
            pl.BlockSpec((1, NN, HID), lambda b, t: (b, 0, 0)),
            full_stats, full_stats,
            pl.BlockSpec((1, HID), lambda b, t: (0, 0)),
            pl.BlockSpec((1, HID), lambda b, t: (0, 0)),
            pl.BlockSpec((1, SEQ, 1), lambda b, t: (b, 0, 0)),
            pl.BlockSpec((1, SEQ, SEQ), lambda b, t: (b, 0, 0)),
            pl.BlockSpec((SEQ, EOD), lambda b, t: (0, 0)),
            pl.BlockSpec((1, 1, TED), lambda b, t: (t[b], 0, 0)),
            pl.BlockSpec((HID, EOD), lambda b, t: (0, 0)),
            pl.BlockSpec((1, EOD), lambda b, t: (0, 0)),
            pl.BlockSpec((ATT, ATT), lambda b, t: (0, 0)),
            pl.BlockSpec((1, ATT), lambda b, t: (0, 0)),
            pl.BlockSpec((ATT, ATT), lambda b, t: (0, 0)),
            pl.BlockSpec((1, ATT), lambda b, t: (0, 0)),
            pl.BlockSpec((ATT, ATT), lambda b, t: (0, 0)),
            pl.BlockSpec((1, ATT), lambda b, t: (0, 0)),
            pl.BlockSpec((ATT, ATT), lambda b, t: (0, 0)),
            pl.BlockSpec((1, ATT), lambda b, t: (0, 0)),
        ],
        out_specs=pl.BlockSpec((1, SEQ, ATT), lambda b, t: (b, 0, 0)),
    )
    return pl.pallas_call(_kc_body, grid_spec=grid_spec,
                          out_shape=jax.ShapeDtypeStruct((B, SEQ, ATT), f32))(
        tidx, e1, etmp1, se1, sse1,
        ge.reshape(1, HID), be.reshape(1, HID),
        route, mask, pe, emb_time.reshape(24, 1, TED),
        Wel, bel.reshape(1, EOD),
        Wq, bq.reshape(1, ATT), Wk, bk.reshape(1, ATT),
        Wv, bv.reshape(1, ATT), Wd, bd.reshape(1, ATT))


def kernel(f, route, mask, edge, node, A, W_node, emb_edges, W_ev, We_U,
           be_U, We_V, be_V, Wn_U, bn_U, Wn_V, bn_V, g_e, b_e, g_n, b_n,
           W_el, b_el, emb_time, Wq, bq, Wk, bk, Wv, bv, Wd, bd):
    ti = f[:, 0].astype(jnp.int32)
    tidx = f[:, 1].astype(jnp.int32)
    route = route.astype(jnp.int32)
    ridx = (route[:, :, 0] * N + route[:, :, 1]).reshape(B, SEQ, 1)
    mask = mask.astype(jnp.int32)
    A3 = A.reshape(T, NN, 1)

    e0, etmp0, x0, xtmp0, se, sse, sx, ssx = _run_ka(
        ti, edge, node, A3, emb_edges, W_ev, W_node,
        We_U[0], be_U[0], We_V[0], be_V[0], Wn_U[0], bn_U[0],
        Wn_V[0], bn_V[0])

    e1, etmp1, se1, sse1 = _run_kb(
        e0, etmp0, x0, xtmp0, se, sse, sx, ssx,
        g_e[0], b_e[0], g_n[0], b_n[0], We_U[1], be_U[1], We_V[1], be_V[1])

    out = _run_kc(
        tidx, e1, etmp1, se1, sse1, g_e[1], b_e[1], ridx, mask,
        _pe_const(), emb_time, W_el, b_el, Wq, bq, Wk, bk, Wv, bv, Wd, bd)

    return jnp.concatenate([out.reshape(B, SEQ * ATT), f[:, 1:]], axis=1)


# speedup vs baseline: 2.2242x; 1.1136x over previous
import math

import jax
import jax.numpy as jnp
import numpy as np
from jax.experimental import pallas as pl
from jax.experimental.pallas import tpu as pltpu

B = 16; T = 24; N = 64; SEQ = 25
HID = 128; NODE_DIM = 8; VOC = 3; EVD = 5; TED = 16; EOD = 24; NL = 2
ATT = 64; H = 8; DH = ATT // H
NN = N * N
EPS = 1e-5


def _pe_const():
    pe = []
    for pos in range(SEQ):
        row = []
        for ii in range(0, EOD, 2):
            row.append(math.sin(pos / 10000 ** (2 * ii / EOD)))
            row.append(math.cos(pos / 10000 ** (2 * ii / EOD)))
        pe.append(row)
    return jnp.asarray(np.array(pe, dtype=np.float32))


def _ka_body(ti_ref, edge_ref, node_ref, A_ref, emb_edges_ref, W_ev_ref,
             W_node_ref, WeU_ref, beU_ref, WeV_ref, beV_ref, WnU_ref,
             bnU_ref, WnV_ref, bnV_ref,
             e0_ref, etmp0_ref, x0_ref, xtmp0_ref, se_ref, sse_ref,
             sx_ref, ssx_ref):
    edge_b = edge_ref[0]
    ev = edge_b[:, :, 2:].reshape(NN, EVD)
    e_vals = jnp.dot(ev, W_ev_ref[...], preferred_element_type=jnp.float32)
    A_b = A_ref[0]
    emb = emb_edges_ref[...]
    e_tags = ((A_b == 0).astype(jnp.float32) * emb[0:1, :]
              + (A_b == 1).astype(jnp.float32) * emb[1:2, :]
              + (A_b == 2).astype(jnp.float32) * emb[2:3, :])
    e0 = jnp.concatenate([e_vals, e_tags], axis=1)
    e0_ref[0] = e0
    x0 = jnp.dot(node_ref[0][:, 1:], W_node_ref[...],
                 preferred_element_type=jnp.float32)
    x0_ref[0] = x0

    Ue = jnp.dot(e0, WeU_ref[...], preferred_element_type=jnp.float32) \
        + beU_ref[...]
    Vx = jnp.dot(x0, WeV_ref[...], preferred_element_type=jnp.float32) \
        + beV_ref[...]
    Ue3 = Ue.reshape(N, N, HID)
    e_tmp = Ue3 + Vx[None, :, :] + Vx[:, None, :]
    e_tmp_f = e_tmp.reshape(NN, HID)
    etmp0_ref[0] = e_tmp_f
    se_ref[0] = jnp.sum(e_tmp_f, axis=0, keepdims=True)
    sse_ref[0] = jnp.sum(e_tmp_f * e_tmp_f, axis=0, keepdims=True)

    gate = jax.nn.sigmoid(e_tmp)
    Vx2 = jnp.dot(x0, WnV_ref[...], preferred_element_type=jnp.float32) \
        + bnV_ref[...]
    num = jnp.sum(gate * Vx2[None, :, :], axis=1)
    den = jnp.sum(gate, axis=1)
    Ux = jnp.dot(x0, WnU_ref[...], preferred_element_type=jnp.float32) \
        + bnU_ref[...]
    x_tmp = Ux + num / (1e-20 + den)
    xtmp0_ref[0] = x_tmp
    sx_ref[0] = jnp.sum(x_tmp, axis=0, keepdims=True)
    ssx_ref[0] = jnp.sum(x_tmp * x_tmp, axis=0, keepdims=True)


def _run_ka(ti, edge, node, A, emb_edges, W_ev, W_node, WeU, beU, WeV, beV,
            WnU, bnU, WnV, bnV):
    f32 = jnp.float32
    grid_spec = pltpu.PrefetchScalarGridSpec(
        num_scalar_prefetch=1,
        grid=(B,),
        in_specs=[
            pl.BlockSpec((1, N, N, EVD + 2), lambda b, ti: (ti[b], 0, 0, 0)),
            pl.BlockSpec((1, N, NODE_DIM + 1), lambda b, ti: (ti[b], 0, 0)),
            pl.BlockSpec((1, NN, 1), lambda b, ti: (ti[b], 0, 0)),
            pl.BlockSpec((VOC, HID // 2), lambda b, ti: (0, 0)),
            pl.BlockSpec((EVD, HID // 2), lambda b, ti: (0, 0)),
            pl.BlockSpec((NODE_DIM, HID), lambda b, ti: (0, 0)),
            pl.BlockSpec((HID, HID), lambda b, ti: (0, 0)),
            pl.BlockSpec((1, HID), lambda b, ti: (0, 0)),
            pl.BlockSpec((HID, HID), lambda b, ti: (0, 0)),
            pl.BlockSpec((1, HID), lambda b, ti: (0, 0)),
            pl.BlockSpec((HID, HID), lambda b, ti: (0, 0)),
            pl.BlockSpec((1, HID), lambda b, ti: (0, 0)),
            pl.BlockSpec((HID, HID), lambda b, ti: (0, 0)),
            pl.BlockSpec((1, HID), lambda b, ti: (0, 0)),
        ],
        out_specs=[
            pl.BlockSpec((1, NN, HID), lambda b, ti: (b, 0, 0)),
            pl.BlockSpec((1, NN, HID), lambda b, ti: (b, 0, 0)),
            pl.BlockSpec((1, N, HID), lambda b, ti: (b, 0, 0)),
            pl.BlockSpec((1, N, HID), lambda b, ti: (b, 0, 0)),
            pl.BlockSpec((1, 1, HID), lambda b, ti: (b, 0, 0)),
            pl.BlockSpec((1, 1, HID), lambda b, ti: (b, 0, 0)),
            pl.BlockSpec((1, 1, HID), lambda b, ti: (b, 0, 0)),
            pl.BlockSpec((1, 1, HID), lambda b, ti: (b, 0, 0)),
        ],
    )
    out_shape = [
        jax.ShapeDtypeStruct((B, NN, HID), f32),
        jax.ShapeDtypeStruct((B, NN, HID), f32),
        jax.ShapeDtypeStruct((B, N, HID), f32),
        jax.ShapeDtypeStruct((B, N, HID), f32),
        jax.ShapeDtypeStruct((B, 1, HID), f32),
        jax.ShapeDtypeStruct((B, 1, HID), f32),
        jax.ShapeDtypeStruct((B, 1, HID), f32),
        jax.ShapeDtypeStruct((B, 1, HID), f32),
    ]
    return pl.pallas_call(_ka_body, grid_spec=grid_spec,
                          out_shape=out_shape,
                          compiler_params=pltpu.CompilerParams(
                              dimension_semantics=("parallel",)))(
        ti, edge, node, A, emb_edges, W_ev, W_node,
        WeU, beU.reshape(1, HID), WeV, beV.reshape(1, HID),
        WnU, bnU.reshape(1, HID), WnV, bnV.reshape(1, HID))


def _kb_body(e0_ref, etmp0_ref, x0_ref, xtmp0_ref,
             se_ref, sse_ref, sx_ref, ssx_ref,
             ge_ref, be_ref, gn_ref, bn_ref,
             WeU_ref, beU_ref, WeV_ref, beV_ref, ridx_ref,
             re1_ref, retmp1_ref, se1_ref, sse1_ref):
    cnt_e = float(B * NN)
    cnt_x = float(B * N)
    m_e = jnp.sum(se_ref[...], axis=0) / cnt_e
    v_e = jnp.sum(sse_ref[...], axis=0) / cnt_e - m_e * m_e
    m_x = jnp.sum(sx_ref[...], axis=0) / cnt_x
    v_x = jnp.sum(ssx_ref[...], axis=0) / cnt_x - m_x * m_x

    x_tmp = xtmp0_ref[0]
    xn = gn_ref[...] * (x_tmp - m_x) * jax.lax.rsqrt(v_x + EPS) + bn_ref[...]
    x1 = x0_ref[0] + jnp.maximum(xn, 0.0)

    e_tmp = etmp0_ref[0]
    en = ge_ref[...] * (e_tmp - m_e) * jax.lax.rsqrt(v_e + EPS) + be_ref[...]
    e1 = e0_ref[0] + jnp.maximum(en, 0.0)

    Ue = jnp.dot(e1, WeU_ref[...], preferred_element_type=jnp.float32) \
        + beU_ref[...]
    Vx = jnp.dot(x1, WeV_ref[...], preferred_element_type=jnp.float32) \
        + beV_ref[...]
    e_tmp1 = Ue.reshape(N, N, HID) + Vx[None, :, :] + Vx[:, None, :]
    e_tmp1_f = e_tmp1.reshape(NN, HID)
    se1_ref[0] = jnp.sum(e_tmp1_f, axis=0, keepdims=True)
    sse1_ref[0] = jnp.sum(e_tmp1_f * e_tmp1_f, axis=0, keepdims=True)

    onehot = (ridx_ref[0] == jax.lax.broadcasted_iota(
        jnp.int32, (SEQ, NN), 1)).astype(jnp.float32)
    re1_ref[0] = jnp.dot(onehot, e1, preferred_element_type=jnp.float32)
    retmp1_ref[0] = jnp.dot(onehot, e_tmp1_f,
                            preferred_element_type=jnp.float32)


def _run_kb(e0, etmp0, x0, xtmp0, se, sse, sx, ssx,
            ge, be, gn, bn, WeU, beU, WeV, beV, ridx):
    f32 = jnp.float32
    full_stats = pl.BlockSpec((B, 1, HID), lambda b: (0, 0, 0))
    grid_spec = pl.GridSpec(
        grid=(B,),
        in_specs=[
            pl.BlockSpec((1, NN, HID), lambda b: (b, 0, 0)),
            pl.BlockSpec((1, NN, HID), lambda b: (b, 0, 0)),
            pl.BlockSpec((1, N, HID), lambda b: (b, 0, 0)),
            pl.BlockSpec((1, N, HID), lambda b: (b, 0, 0)),
            full_stats, full_stats, full_stats, full_stats,
            pl.BlockSpec((1, HID), lambda b: (0, 0)),
            pl.BlockSpec((1, HID), lambda b: (0, 0)),
            pl.BlockSpec((1, HID), lambda b: (0, 0)),
            pl.BlockSpec((1, HID), lambda b: (0, 0)),
            pl.BlockSpec((HID, HID), lambda b: (0, 0)),
            pl.BlockSpec((1, HID), lambda b: (0, 0)),
            pl.BlockSpec((HID, HID), lambda b: (0, 0)),
            pl.BlockSpec((1, HID), lambda b: (0, 0)),
            pl.BlockSpec((1, SEQ, 1), lambda b: (b, 0, 0)),
        ],
        out_specs=[
            pl.BlockSpec((1, SEQ, HID), lambda b: (b, 0, 0)),
            pl.BlockSpec((1, SEQ, HID), lambda b: (b, 0, 0)),
            pl.BlockSpec((1, 1, HID), lambda b: (b, 0, 0)),
            pl.BlockSpec((1, 1, HID), lambda b: (b, 0, 0)),
        ],
    )
    out_shape = [
        jax.ShapeDtypeStruct((B, SEQ, HID), f32),
        jax.ShapeDtypeStruct((B, SEQ, HID), f32),
        jax.ShapeDtypeStruct((B, 1, HID), f32),
        jax.ShapeDtypeStruct((B, 1, HID), f32),
    ]
    return pl.pallas_call(_kb_body, grid_spec=grid_spec,
                          out_shape=out_shape,
                          compiler_params=pltpu.CompilerParams(
                              dimension_semantics=("parallel",)))(
        e0, etmp0, x0, xtmp0, se, sse, sx, ssx,
        ge.reshape(1, HID), be.reshape(1, HID),
        gn.reshape(1, HID), bn.reshape(1, HID),
        WeU, beU.reshape(1, HID), WeV, beV.reshape(1, HID), ridx)


def _kc_body(tidx_ref, re1_ref, retmp1_ref, se1_ref, sse1_ref,
             ge_ref, be_ref, mask_ref, pe_ref, emb_time_ref,
             Wel_ref, bel_ref, Wq_ref, bq_ref, Wk_ref, bk_ref,
             Wv_ref, bv_ref, Wd_ref, bd_ref, out_ref):
    cnt_e = float(B * NN)
    m_e = jnp.sum(se1_ref[...], axis=0) / cnt_e
    v_e = jnp.sum(sse1_ref[...], axis=0) / cnt_e - m_e * m_e

    e_tmp = retmp1_ref[0]
    en = ge_ref[...] * (e_tmp - m_e) * jax.lax.rsqrt(v_e + EPS) + be_ref[...]
    rows = re1_ref[0] + jnp.maximum(en, 0.0)
    r_edge = jnp.dot(rows, Wel_ref[...],
                     preferred_element_type=jnp.float32) + bel_ref[...]
    t_emb = jnp.broadcast_to(emb_time_ref[0, 0].reshape(1, TED), (SEQ, TED))
    R = jnp.concatenate([r_edge, pe_ref[...], t_emb], axis=1)

    q = jnp.dot(R, Wq_ref[...], preferred_element_type=jnp.float32) \
        + bq_ref[...]
    k = jnp.dot(R, Wk_ref[...], preferred_element_type=jnp.float32) \
        + bk_ref[...]
    v = jnp.dot(R, Wv_ref[...], preferred_element_type=jnp.float32) \
        + bv_ref[...]
    mask0 = mask_ref[0] == 0
    scale = 1.0 / math.sqrt(DH)
    ctx_heads = []
    for h in range(H):
        sl = slice(h * DH, (h + 1) * DH)
        qh = q[:, sl]
        kh = k[:, sl]
        vh = v[:, sl]
        s = jax.lax.dot_general(qh, kh, (((1,), (1,)), ((), ())),
                                preferred_element_type=jnp.float32) * scale
        s = jnp.where(mask0, 1e-8, s)
        s = s - jnp.max(s, axis=1, keepdims=True)
        p = jnp.exp(s)
        p = p / jnp.sum(p, axis=1, keepdims=True)
        ctx_heads.append(jnp.dot(p, vh, preferred_element_type=jnp.float32))
    ctx = jnp.concatenate(ctx_heads, axis=1)
    out_ref[0] = jnp.dot(ctx, Wd_ref[...],
                         preferred_element_type=jnp.float32) + bd_ref[...]


def _run_kc(tidx, re1, retmp1, se1, sse1, ge, be, mask, pe, emb_time,
            Wel, bel, Wq, bq, Wk, bk, Wv, bv, Wd, bd):
    f32 = jnp.float32
    full_stats = pl.BlockSpec((B, 1, HID), lambda b, t: (0, 0, 0))
    grid_spec = pltpu.PrefetchScalarGridSpec(
        num_scalar_prefetch=1,
        grid=(B,),
        in_specs=[
            pl.BlockSpec((1, SEQ, HID), lambda b, t: (b, 0, 0)),
            pl.BlockSpec((1, SEQ, HID), lambda b, t: (b, 0, 0)),
            full_stats, full_stats,
            pl.BlockSpec((1, HID), lambda b, t: (0, 0)),
            pl.BlockSpec((1, HID), lambda b, t: (0, 0)),
            pl.BlockSpec((1, SEQ, SEQ), lambda b, t: (b, 0, 0)),
            pl.BlockSpec((SEQ, EOD), lambda b, t: (0, 0)),
            pl.BlockSpec((1, 1, TED), lambda b, t: (t[b], 0, 0)),
            pl.BlockSpec((HID, EOD), lambda b, t: (0, 0)),
            pl.BlockSpec((1, EOD), lambda b, t: (0, 0)),
            pl.BlockSpec((ATT, ATT), lambda b, t: (0, 0)),
            pl.BlockSpec((1, ATT), lambda b, t: (0, 0)),
            pl.BlockSpec((ATT, ATT), lambda b, t: (0, 0)),
            pl.BlockSpec((1, ATT), lambda b, t: (0, 0)),
            pl.BlockSpec((ATT, ATT), lambda b, t: (0, 0)),
            pl.BlockSpec((1, ATT), lambda b, t: (0, 0)),
            pl.BlockSpec((ATT, ATT), lambda b, t: (0, 0)),
            pl.BlockSpec((1, ATT), lambda b, t: (0, 0)),
        ],
        out_specs=pl.BlockSpec((1, SEQ, ATT), lambda b, t: (b, 0, 0)),
    )
    return pl.pallas_call(_kc_body, grid_spec=grid_spec,
                          out_shape=jax.ShapeDtypeStruct((B, SEQ, ATT), f32),
                          compiler_params=pltpu.CompilerParams(
                              dimension_semantics=("parallel",)))(
        tidx, re1, retmp1, se1, sse1,
        ge.reshape(1, HID), be.reshape(1, HID),
        mask, pe, emb_time.reshape(24, 1, TED),
        Wel, bel.reshape(1, EOD),
        Wq, bq.reshape(1, ATT), Wk, bk.reshape(1, ATT),
        Wv, bv.reshape(1, ATT), Wd, bd.reshape(1, ATT))


def kernel(f, route, mask, edge, node, A, W_node, emb_edges, W_ev, We_U,
           be_U, We_V, be_V, Wn_U, bn_U, Wn_V, bn_V, g_e, b_e, g_n, b_n,
           W_el, b_el, emb_time, Wq, bq, Wk, bk, Wv, bv, Wd, bd):
    ti = f[:, 0].astype(jnp.int32)
    tidx = f[:, 1].astype(jnp.int32)
    route = route.astype(jnp.int32)
    ridx = (route[:, :, 0] * N + route[:, :, 1]).reshape(B, SEQ, 1)
    mask = mask.astype(jnp.int32)
    A3 = A.reshape(T, NN, 1)

    e0, etmp0, x0, xtmp0, se, sse, sx, ssx = _run_ka(
        ti, edge, node, A3, emb_edges, W_ev, W_node,
        We_U[0], be_U[0], We_V[0], be_V[0], Wn_U[0], bn_U[0],
        Wn_V[0], bn_V[0])

    re1, retmp1, se1, sse1 = _run_kb(
        e0, etmp0, x0, xtmp0, se, sse, sx, ssx,
        g_e[0], b_e[0], g_n[0], b_n[0], We_U[1], be_U[1], We_V[1], be_V[1],
        ridx)

    out = _run_kc(
        tidx, re1, retmp1, se1, sse1, g_e[1], b_e[1], mask,
        _pe_const(), emb_time, W_el, b_el, Wq, bq, Wk, bk, Wv, bv, Wd, bd)

    return jnp.concatenate([out.reshape(B, SEQ * ATT), f[:, 1:]], axis=1)


# speedup vs baseline: 2.8195x; 1.2676x over previous
import math

import jax
import jax.numpy as jnp
import numpy as np
from jax.experimental import pallas as pl
from jax.experimental.pallas import tpu as pltpu

B = 16; T = 24; N = 64; SEQ = 25
HID = 128; NODE_DIM = 8; VOC = 3; EVD = 5; TED = 16; EOD = 24; NL = 2
ATT = 64; H = 8; DH = ATT // H
NN = N * N
HS = H * SEQ
EPS = 1e-5


def _pe_const():
    pe = []
    for pos in range(SEQ):
        row = []
        for ii in range(0, EOD, 2):
            row.append(math.sin(pos / 10000 ** (2 * ii / EOD)))
            row.append(math.cos(pos / 10000 ** (2 * ii / EOD)))
        pe.append(row)
    return jnp.asarray(np.array(pe, dtype=np.float32))


def _hmask_const():
    m = np.zeros((HS, ATT), dtype=np.float32)
    for h in range(H):
        m[h * SEQ:(h + 1) * SEQ, h * DH:(h + 1) * DH] = 1.0
    return jnp.asarray(m)


def _embed(ev, A_b, emb, W_ev):
    e_vals = jnp.dot(ev, W_ev, preferred_element_type=jnp.float32)
    onehot_a = (A_b == jax.lax.broadcasted_iota(jnp.int32, (NN, VOC), 1)
                ).astype(jnp.float32)
    e_tags = jnp.dot(onehot_a, emb, preferred_element_type=jnp.float32)
    return jnp.concatenate([e_vals, e_tags], axis=1), onehot_a


def _ka_body(ti_ref, ev_ref, node_ref, A_ref, emb_edges_ref, W_ev_ref,
             W_node_ref, WeU_ref, beU_ref, WeV_ref, beV_ref, WnU_ref,
             bnU_ref, WnV_ref, bnV_ref,
             x0_ref, xtmp0_ref, se_ref, sse_ref, sx_ref, ssx_ref):
    ev = ev_ref[0]
    A_b = A_ref[0]
    x0 = jnp.dot(node_ref[0][:, 1:], W_node_ref[...],
                 preferred_element_type=jnp.float32)
    x0_ref[0] = x0

    W1 = jnp.dot(W_ev_ref[...], WeU_ref[:HID // 2, :],
                 preferred_element_type=jnp.float32)
    W2 = jnp.dot(emb_edges_ref[...], WeU_ref[HID // 2:, :],
                 preferred_element_type=jnp.float32)
    onehot_a = (A_b == jax.lax.broadcasted_iota(jnp.int32, (NN, VOC), 1)
                ).astype(jnp.float32)
    Ue = (jnp.dot(ev, W1, preferred_element_type=jnp.float32)
          + jnp.dot(onehot_a, W2, preferred_element_type=jnp.float32))
    Vx = jnp.dot(x0, WeV_ref[...], preferred_element_type=jnp.float32) \
        + beV_ref[...] + beU_ref[...]
    Vxi = Vx - beU_ref[...]
    e_tmp = Ue.reshape(N, N, HID) + Vx[None, :, :] + Vxi[:, None, :]
    e_tmp_f = e_tmp.reshape(NN, HID)
    se_ref[0] = jnp.sum(e_tmp_f, axis=0, keepdims=True)
    sse_ref[0] = jnp.sum(e_tmp_f * e_tmp_f, axis=0, keepdims=True)

    gate = jax.nn.sigmoid(e_tmp)
    Vx2 = jnp.dot(x0, WnV_ref[...], preferred_element_type=jnp.float32) \
        + bnV_ref[...]
    num = jnp.sum(gate * Vx2[None, :, :], axis=1)
    den = jnp.sum(gate, axis=1)
    Ux = jnp.dot(x0, WnU_ref[...], preferred_element_type=jnp.float32) \
        + bnU_ref[...]
    x_tmp = Ux + num / (1e-20 + den)
    xtmp0_ref[0] = x_tmp
    sx_ref[0] = jnp.sum(x_tmp, axis=0, keepdims=True)
    ssx_ref[0] = jnp.sum(x_tmp * x_tmp, axis=0, keepdims=True)


def _run_ka(ti, edge_v, node, A, emb_edges, W_ev, W_node, WeU, beU, WeV,
            beV, WnU, bnU, WnV, bnV):
    f32 = jnp.float32
    grid_spec = pltpu.PrefetchScalarGridSpec(
        num_scalar_prefetch=1,
        grid=(B,),
        in_specs=[
            pl.BlockSpec((1, NN, EVD), lambda b, ti: (ti[b], 0, 0)),
            pl.BlockSpec((1, N, NODE_DIM + 1), lambda b, ti: (ti[b], 0, 0)),
            pl.BlockSpec((1, NN, 1), lambda b, ti: (ti[b], 0, 0)),
            pl.BlockSpec((VOC, HID // 2), lambda b, ti: (0, 0)),
            pl.BlockSpec((EVD, HID // 2), lambda b, ti: (0, 0)),
            pl.BlockSpec((NODE_DIM, HID), lambda b, ti: (0, 0)),
            pl.BlockSpec((HID, HID), lambda b, ti: (0, 0)),
            pl.BlockSpec((1, HID), lambda b, ti: (0, 0)),
            pl.BlockSpec((HID, HID), lambda b, ti: (0, 0)),
            pl.BlockSpec((1, HID), lambda b, ti: (0, 0)),
            pl.BlockSpec((HID, HID), lambda b, ti: (0, 0)),
            pl.BlockSpec((1, HID), lambda b, ti: (0, 0)),
            pl.BlockSpec((HID, HID), lambda b, ti: (0, 0)),
            pl.BlockSpec((1, HID), lambda b, ti: (0, 0)),
        ],
        out_specs=[
            pl.BlockSpec((1, N, HID), lambda b, ti: (b, 0, 0)),
            pl.BlockSpec((1, N, HID), lambda b, ti: (b, 0, 0)),
            pl.BlockSpec((1, 1, HID), lambda b, ti: (b, 0, 0)),
            pl.BlockSpec((1, 1, HID), lambda b, ti: (b, 0, 0)),
            pl.BlockSpec((1, 1, HID), lambda b, ti: (b, 0, 0)),
            pl.BlockSpec((1, 1, HID), lambda b, ti: (b, 0, 0)),
        ],
    )
    out_shape = [
        jax.ShapeDtypeStruct((B, N, HID), f32),
        jax.ShapeDtypeStruct((B, N, HID), f32),
        jax.ShapeDtypeStruct((B, 1, HID), f32),
        jax.ShapeDtypeStruct((B, 1, HID), f32),
        jax.ShapeDtypeStruct((B, 1, HID), f32),
        jax.ShapeDtypeStruct((B, 1, HID), f32),
    ]
    return pl.pallas_call(_ka_body, grid_spec=grid_spec,
                          out_shape=out_shape,
                          compiler_params=pltpu.CompilerParams(
                              dimension_semantics=("parallel",)))(
        ti, edge_v, node, A, emb_edges, W_ev, W_node,
        WeU, beU.reshape(1, HID), WeV, beV.reshape(1, HID),
        WnU, bnU.reshape(1, HID), WnV, bnV.reshape(1, HID))


def _kb_body(ti_ref, ev_ref, A_ref, emb_edges_ref, W_ev_ref,
             x0_ref, xtmp0_ref, se_ref, sse_ref, sx_ref, ssx_ref,
             ge_ref, be_ref, gn_ref, bn_ref,
             WeU0_ref, beU0_ref, WeV0_ref, beV0_ref,
             WeU_ref, beU_ref, WeV_ref, beV_ref, ridx_ref,
             re1_ref, retmp1_ref, se1_ref, sse1_ref):
    cnt_e = float(B * NN)
    cnt_x = float(B * N)
    m_e = jnp.sum(se_ref[...], axis=0) / cnt_e
    v_e = jnp.sum(sse_ref[...], axis=0) / cnt_e - m_e * m_e
    m_x = jnp.sum(sx_ref[...], axis=0) / cnt_x
    v_x = jnp.sum(ssx_ref[...], axis=0) / cnt_x - m_x * m_x
    sc_e = ge_ref[...] * jax.lax.rsqrt(v_e + EPS)
    sh_e = be_ref[...] - m_e * sc_e
    sc_x = gn_ref[...] * jax.lax.rsqrt(v_x + EPS)
    sh_x = bn_ref[...] - m_x * sc_x

    x0 = x0_ref[0]
    x1 = x0 + jnp.maximum(xtmp0_ref[0] * sc_x + sh_x, 0.0)

    e0, _ = _embed(ev_ref[0], A_ref[0], emb_edges_ref[...], W_ev_ref[...])
    Ue0 = jnp.dot(e0, WeU0_ref[...], preferred_element_type=jnp.float32)
    Vx0 = jnp.dot(x0, WeV0_ref[...], preferred_element_type=jnp.float32) \
        + beV0_ref[...] + beU0_ref[...]
    Vx0i = Vx0 - beU0_ref[...]
    e_tmp0 = Ue0.reshape(N, N, HID) + Vx0[None, :, :] + Vx0i[:, None, :]
    e_tmp0_f = e_tmp0.reshape(NN, HID)
    e1 = e0 + jnp.maximum(e_tmp0_f * sc_e + sh_e, 0.0)

    Ue = jnp.dot(e1, WeU_ref[...], preferred_element_type=jnp.float32)
    Vx = jnp.dot(x1, WeV_ref[...], preferred_element_type=jnp.float32) \
        + beV_ref[...] + beU_ref[...]
    Vxi = Vx - beU_ref[...]
    e_tmp1 = Ue.reshape(N, N, HID) + Vx[None, :, :] + Vxi[:, None, :]
    e_tmp1_f = e_tmp1.reshape(NN, HID)
    se1_ref[0] = jnp.sum(e_tmp1_f, axis=0, keepdims=True)
    sse1_ref[0] = jnp.sum(e_tmp1_f * e_tmp1_f, axis=0, keepdims=True)

    onehot = (ridx_ref[0] == jax.lax.broadcasted_iota(
        jnp.int32, (SEQ, NN), 1)).astype(jnp.float32)
    re1_ref[0] = jnp.dot(onehot, e1, preferred_element_type=jnp.float32)
    retmp1_ref[0] = jnp.dot(onehot, e_tmp1_f,
                            preferred_element_type=jnp.float32)


def _run_kb(ti, edge_v, A, emb_edges, W_ev, x0, xtmp0, se, sse, sx, ssx,
            ge, be, gn, bn, WeU0, beU0, WeV0, beV0, WeU, beU, WeV, beV,
            ridx):
    f32 = jnp.float32
    full_stats = pl.BlockSpec((B, 1, HID), lambda b, ti: (0, 0, 0))
    w_spec = pl.BlockSpec((HID, HID), lambda b, ti: (0, 0))
    b_spec = pl.BlockSpec((1, HID), lambda b, ti: (0, 0))
    grid_spec = pltpu.PrefetchScalarGridSpec(
        num_scalar_prefetch=1,
        grid=(B,),
        in_specs=[
            pl.BlockSpec((1, NN, EVD), lambda b, ti: (ti[b], 0, 0)),
            pl.BlockSpec((1, NN, 1), lambda b, ti: (ti[b], 0, 0)),
            pl.BlockSpec((VOC, HID // 2), lambda b, ti: (0, 0)),
            pl.BlockSpec((EVD, HID // 2), lambda b, ti: (0, 0)),
            pl.BlockSpec((1, N, HID), lambda b, ti: (b, 0, 0)),
            pl.BlockSpec((1, N, HID), lambda b, ti: (b, 0, 0)),
            full_stats, full_stats, full_stats, full_stats,
            b_spec, b_spec, b_spec, b_spec,
            w_spec, b_spec, w_spec, b_spec,
            w_spec, b_spec, w_spec, b_spec,
            pl.BlockSpec((1, SEQ, 1), lambda b, ti: (b, 0, 0)),
        ],
        out_specs=[
            pl.BlockSpec((1, SEQ, HID), lambda b, ti: (b, 0, 0)),
            pl.BlockSpec((1, SEQ, HID), lambda b, ti: (b, 0, 0)),
            pl.BlockSpec((1, 1, HID), lambda b, ti: (b, 0, 0)),
            pl.BlockSpec((1, 1, HID), lambda b, ti: (b, 0, 0)),
        ],
    )
    out_shape = [
        jax.ShapeDtypeStruct((B, SEQ, HID), f32),
        jax.ShapeDtypeStruct((B, SEQ, HID), f32),
        jax.ShapeDtypeStruct((B, 1, HID), f32),
        jax.ShapeDtypeStruct((B, 1, HID), f32),
    ]
    return pl.pallas_call(_kb_body, grid_spec=grid_spec,
                          out_shape=out_shape,
                          compiler_params=pltpu.CompilerParams(
                              dimension_semantics=("parallel",)))(
        ti, edge_v, A, emb_edges, W_ev, x0, xtmp0, se, sse, sx, ssx,
        ge.reshape(1, HID), be.reshape(1, HID),
        gn.reshape(1, HID), bn.reshape(1, HID),
        WeU0, beU0.reshape(1, HID), WeV0, beV0.reshape(1, HID),
        WeU, beU.reshape(1, HID), WeV, beV.reshape(1, HID), ridx)


def _kc_body(tidx_ref, re1_ref, retmp1_ref, se1_ref, sse1_ref,
             ge_ref, be_ref, mask_ref, pe_ref, emb_time_ref, hmask_ref,
             Wel_ref, bel_ref, Wq_ref, bq_ref, Wk_ref, bk_ref,
             Wv_ref, bv_ref, Wd_ref, bd_ref, out_ref):
    cnt_e = float(B * NN)
    m_e = jnp.sum(se1_ref[...], axis=0) / cnt_e
    v_e = jnp.sum(sse1_ref[...], axis=0) / cnt_e - m_e * m_e
    sc_e = ge_ref[...] * jax.lax.rsqrt(v_e + EPS)
    sh_e = be_ref[...] - m_e * sc_e

    t_onehot = (tidx_ref[...] == jax.lax.broadcasted_iota(
        jnp.int32, (B, T), 1)).astype(jnp.float32)
    temb_all = jnp.dot(t_onehot, emb_time_ref[...],
                       preferred_element_type=jnp.float32)
    hmask = hmask_ref[...]
    scale = 1.0 / math.sqrt(DH)

    for b in range(B):
        rows = re1_ref[b] + jnp.maximum(
            retmp1_ref[b] * sc_e + sh_e, 0.0)
        r_edge = jnp.dot(rows, Wel_ref[...],
                         preferred_element_type=jnp.float32) + bel_ref[...]
        t_emb = jnp.broadcast_to(temb_all[b:b + 1, :], (SEQ, TED))
        R = jnp.concatenate([r_edge, pe_ref[...], t_emb], axis=1)

        q = jnp.dot(R, Wq_ref[...], preferred_element_type=jnp.float32) \
            + bq_ref[...]
        k = jnp.dot(R, Wk_ref[...], preferred_element_type=jnp.float32) \
            + bk_ref[...]
        v = jnp.dot(R, Wv_ref[...], preferred_element_type=jnp.float32) \
            + bv_ref[...]
        q_blk = jnp.tile(q, (H, 1)) * hmask
        s = jax.lax.dot_general(q_blk, k, (((1,), (1,)), ((), ())),
                                preferred_element_type=jnp.float32) * scale
        mtile = jnp.tile(mask_ref[b], (H, 1)) == 0
        s = jnp.where(mtile, 1e-8, s)
        s = s - jnp.max(s, axis=1, keepdims=True)
        p = jnp.exp(s)
        p = p / jnp.sum(p, axis=1, keepdims=True)
        ctx_full = jnp.dot(p, v, preferred_element_type=jnp.float32)
        ctx = jnp.sum((ctx_full * hmask).reshape(H, SEQ, ATT), axis=0)
        out_ref[b] = jnp.dot(ctx, Wd_ref[...],
                             preferred_element_type=jnp.float32) + bd_ref[...]


def _run_kc(tidx, re1, retmp1, se1, sse1, ge, be, mask, pe, emb_time,
            Wel, bel, Wq, bq, Wk, bk, Wv, bv, Wd, bd):
    f32 = jnp.float32
    full = lambda *shape: pl.BlockSpec(shape, lambda: tuple(
        0 for _ in shape))
    grid_spec = pl.GridSpec(
        grid=(),
        in_specs=[
            full(B, 1),
            full(B, SEQ, HID), full(B, SEQ, HID),
            full(B, 1, HID), full(B, 1, HID),
            full(1, HID), full(1, HID),
            full(B, SEQ, SEQ), full(SEQ, EOD), full(T, TED), full(HS, ATT),
            full(HID, EOD), full(1, EOD),
            full(ATT, ATT), full(1, ATT), full(ATT, ATT), full(1, ATT),
            full(ATT, ATT), full(1, ATT), full(ATT, ATT), full(1, ATT),
        ],
        out_specs=full(B, SEQ, ATT),
    )
    return pl.pallas_call(_kc_body, grid_spec=grid_spec,
                          out_shape=jax.ShapeDtypeStruct((B, SEQ, ATT), f32))(
        tidx, re1, retmp1, se1, sse1,
        ge.reshape(1, HID), be.reshape(1, HID),
        mask, pe, emb_time, _hmask_const(),
        Wel, bel.reshape(1, EOD),
        Wq, bq.reshape(1, ATT), Wk, bk.reshape(1, ATT),
        Wv, bv.reshape(1, ATT), Wd, bd.reshape(1, ATT))


def kernel(f, route, mask, edge, node, A, W_node, emb_edges, W_ev, We_U,
           be_U, We_V, be_V, Wn_U, bn_U, Wn_V, bn_V, g_e, b_e, g_n, b_n,
           W_el, b_el, emb_time, Wq, bq, Wk, bk, Wv, bv, Wd, bd):
    ti = f[:, 0].astype(jnp.int32)
    tidx = f[:, 1].astype(jnp.int32).reshape(B, 1)
    route = route.astype(jnp.int32)
    ridx = (route[:, :, 0] * N + route[:, :, 1]).reshape(B, SEQ, 1)
    mask = mask.astype(jnp.int32)
    A3 = A.reshape(T, NN, 1)
    edge_v = edge[:, :, :, 2:].reshape(T, NN, EVD)

    x0, xtmp0, se, sse, sx, ssx = _run_ka(
        ti, edge_v, node, A3, emb_edges, W_ev, W_node,
        We_U[0], be_U[0], We_V[0], be_V[0], Wn_U[0], bn_U[0],
        Wn_V[0], bn_V[0])

    re1, retmp1, se1, sse1 = _run_kb(
        ti, edge_v, A3, emb_edges, W_ev, x0, xtmp0, se, sse, sx, ssx,
        g_e[0], b_e[0], g_n[0], b_n[0],
        We_U[0], be_U[0], We_V[0], be_V[0],
        We_U[1], be_U[1], We_V[1], be_V[1], ridx)

    out = _run_kc(
        tidx, re1, retmp1, se1, sse1, g_e[1], b_e[1], mask,
        _pe_const(), emb_time, W_el, b_el, Wq, bq, Wk, bk, Wv, bv, Wd, bd)

    return jnp.concatenate([out.reshape(B, SEQ * ATT), f[:, 1:]], axis=1)


# speedup vs baseline: 2.8352x; 1.0056x over previous
import math

import jax
import jax.numpy as jnp
import numpy as np
from jax.experimental import pallas as pl
from jax.experimental.pallas import tpu as pltpu

B = 16; T = 24; N = 64; SEQ = 25
HID = 128; NODE_DIM = 8; VOC = 3; EVD = 5; TED = 16; EOD = 24; NL = 2
ATT = 64; H = 8; DH = ATT // H
NN = N * N
HS = H * SEQ
EPS = 1e-5
CNT_E = float(B * NN)
CNT_X = float(B * N)


def _pe_const():
    pe = []
    for pos in range(SEQ):
        row = []
        for ii in range(0, EOD, 2):
            row.append(math.sin(pos / 10000 ** (2 * ii / EOD)))
            row.append(math.cos(pos / 10000 ** (2 * ii / EOD)))
        pe.append(row)
    return jnp.asarray(np.array(pe, dtype=np.float32))


def _hmask_const():
    m = np.zeros((HS, ATT), dtype=np.float32)
    for h in range(H):
        m[h * SEQ:(h + 1) * SEQ, h * DH:(h + 1) * DH] = 1.0
    return jnp.asarray(m)


def _embed(ev, A_b, emb, W_ev):
    e_vals = jnp.dot(ev, W_ev, preferred_element_type=jnp.float32)
    onehot_a = (A_b == jax.lax.broadcasted_iota(jnp.int32, (NN, VOC), 1)
                ).astype(jnp.float32)
    e_tags = jnp.dot(onehot_a, emb, preferred_element_type=jnp.float32)
    return jnp.concatenate([e_vals, e_tags], axis=1), onehot_a


def _body(ti_ref, ev_ref, node_ref, A_ref, emb_edges_ref, W_ev_ref,
          W_node_ref, WeU0_ref, beU0_ref, WeV0_ref, beV0_ref, WnU0_ref,
          bnU0_ref, WnV0_ref, bnV0_ref, ge0_ref, be0_ref, gn0_ref,
          bn0_ref, WeU1_ref, beU1_ref, WeV1_ref, beV1_ref, ridx_ref,
          ge1_ref, be1_ref, tidx_ref, mask_ref, pe_ref, emb_time_ref,
          hmask_ref, Wel_ref, bel_ref, Wq_ref, bq_ref, Wk_ref, bk_ref,
          Wv_ref, bv_ref, Wd_ref, bd_ref,
          out_ref,
          x0_s, xtmp0_s, se_s, sse_s, sx_s, ssx_s,
          re1_s, retmp1_s, se1_s, sse1_s):
    i = pl.program_id(0)

    @pl.when(i == 0)
    def _zero_a():
        z = jnp.zeros((1, HID), jnp.float32)
        se_s[...] = z; sse_s[...] = z; sx_s[...] = z; ssx_s[...] = z

    @pl.when(i < B)
    def _phase_a():
        ev = ev_ref[0]
        A_b = A_ref[0]
        x0 = jnp.dot(node_ref[0][:, 1:], W_node_ref[...],
                     preferred_element_type=jnp.float32)
        x0_s[pl.ds(i * N, N), :] = x0

        W1 = jnp.dot(W_ev_ref[...], WeU0_ref[:HID // 2, :],
                     preferred_element_type=jnp.float32)
        W2 = jnp.dot(emb_edges_ref[...], WeU0_ref[HID // 2:, :],
                     preferred_element_type=jnp.float32)
        onehot_a = (A_b == jax.lax.broadcasted_iota(
            jnp.int32, (NN, VOC), 1)).astype(jnp.float32)
        Ue = (jnp.dot(ev, W1, preferred_element_type=jnp.float32)
              + jnp.dot(onehot_a, W2, preferred_element_type=jnp.float32))
        Vx = jnp.dot(x0, WeV0_ref[...], preferred_element_type=jnp.float32) \
            + beV0_ref[...] + beU0_ref[...]
        Vxi = Vx - beU0_ref[...]
        e_tmp = Ue.reshape(N, N, HID) + Vx[None, :, :] + Vxi[:, None, :]
        e_tmp_f = e_tmp.reshape(NN, HID)
        se_s[...] += jnp.sum(e_tmp_f, axis=0, keepdims=True)
        sse_s[...] += jnp.sum(e_tmp_f * e_tmp_f, axis=0, keepdims=True)

        gate = jax.nn.sigmoid(e_tmp)
        Vx2 = jnp.dot(x0, WnV0_ref[...], preferred_element_type=jnp.float32) \
            + bnV0_ref[...]
        num = jnp.sum(gate * Vx2[None, :, :], axis=1)
        den = jnp.sum(gate, axis=1)
        Ux = jnp.dot(x0, WnU0_ref[...], preferred_element_type=jnp.float32) \
            + bnU0_ref[...]
        x_tmp = Ux + num / (1e-20 + den)
        xtmp0_s[pl.ds(i * N, N), :] = x_tmp
        sx_s[...] += jnp.sum(x_tmp, axis=0, keepdims=True)
        ssx_s[...] += jnp.sum(x_tmp * x_tmp, axis=0, keepdims=True)

    @pl.when(i == B)
    def _zero_b():
        z = jnp.zeros((1, HID), jnp.float32)
        se1_s[...] = z; sse1_s[...] = z

    @pl.when((i >= B) & (i < 2 * B))
    def _phase_b():
        b = i - B
        m_e = se_s[...] / CNT_E
        v_e = sse_s[...] / CNT_E - m_e * m_e
        m_x = sx_s[...] / CNT_X
        v_x = ssx_s[...] / CNT_X - m_x * m_x
        sc_e = ge0_ref[...] * jax.lax.rsqrt(v_e + EPS)
        sh_e = be0_ref[...] - m_e * sc_e
        sc_x = gn0_ref[...] * jax.lax.rsqrt(v_x + EPS)
        sh_x = bn0_ref[...] - m_x * sc_x

        x0 = x0_s[pl.ds(b * N, N), :]
        x1 = x0 + jnp.maximum(xtmp0_s[pl.ds(b * N, N), :] * sc_x + sh_x,
                              0.0)

        e0, _ = _embed(ev_ref[0], A_ref[0], emb_edges_ref[...],
                       W_ev_ref[...])
        Ue0 = jnp.dot(e0, WeU0_ref[...], preferred_element_type=jnp.float32)
        Vx0 = jnp.dot(x0, WeV0_ref[...], preferred_element_type=jnp.float32) \
            + beV0_ref[...] + beU0_ref[...]
        Vx0i = Vx0 - beU0_ref[...]
        e_tmp0 = Ue0.reshape(N, N, HID) + Vx0[None, :, :] + Vx0i[:, None, :]
        e1 = e0 + jnp.maximum(e_tmp0.reshape(NN, HID) * sc_e + sh_e, 0.0)

        Ue1 = jnp.dot(e1, WeU1_ref[...], preferred_element_type=jnp.float32)
        Vx1 = jnp.dot(x1, WeV1_ref[...], preferred_element_type=jnp.float32) \
            + beV1_ref[...] + beU1_ref[...]
        Vx1i = Vx1 - beU1_ref[...]
        e_tmp1 = Ue1.reshape(N, N, HID) + Vx1[None, :, :] + Vx1i[:, None, :]
        e_tmp1_f = e_tmp1.reshape(NN, HID)
        se1_s[...] += jnp.sum(e_tmp1_f, axis=0, keepdims=True)
        sse1_s[...] += jnp.sum(e_tmp1_f * e_tmp1_f, axis=0, keepdims=True)

        onehot = (ridx_ref[0] == jax.lax.broadcasted_iota(
            jnp.int32, (SEQ, NN), 1)).astype(jnp.float32)
        re1_s[pl.ds(b * SEQ, SEQ), :] = jnp.dot(
            onehot, e1, preferred_element_type=jnp.float32)
        retmp1_s[pl.ds(b * SEQ, SEQ), :] = jnp.dot(
            onehot, e_tmp1_f, preferred_element_type=jnp.float32)

    @pl.when(i == 2 * B)
    def _phase_c():
        m_e = se1_s[...] / CNT_E
        v_e = sse1_s[...] / CNT_E - m_e * m_e
        sc_e = ge1_ref[...] * jax.lax.rsqrt(v_e + EPS)
        sh_e = be1_ref[...] - m_e * sc_e

        t_onehot = (tidx_ref[...] == jax.lax.broadcasted_iota(
            jnp.int32, (B, T), 1)).astype(jnp.float32)
        temb_all = jnp.dot(t_onehot, emb_time_ref[...],
                           preferred_element_type=jnp.float32)
        hmask = hmask_ref[...]
        scale = 1.0 / math.sqrt(DH)

        for b in range(B):
            rows = re1_s[b * SEQ:(b + 1) * SEQ, :] + jnp.maximum(
                retmp1_s[b * SEQ:(b + 1) * SEQ, :] * sc_e + sh_e, 0.0)
            r_edge = jnp.dot(rows, Wel_ref[...],
                             preferred_element_type=jnp.float32) \
                + bel_ref[...]
            t_emb = jnp.broadcast_to(temb_all[b:b + 1, :], (SEQ, TED))
            R = jnp.concatenate([r_edge, pe_ref[...], t_emb], axis=1)

            q = jnp.dot(R, Wq_ref[...], preferred_element_type=jnp.float32) \
                + bq_ref[...]
            k = jnp.dot(R, Wk_ref[...], preferred_element_type=jnp.float32) \
                + bk_ref[...]
            v = jnp.dot(R, Wv_ref[...], preferred_element_type=jnp.float32) \
                + bv_ref[...]
            q_blk = jnp.tile(q, (H, 1)) * hmask
            s = jax.lax.dot_general(
                q_blk, k, (((1,), (1,)), ((), ())),
                preferred_element_type=jnp.float32) * scale
            mtile = jnp.tile(mask_ref[b], (H, 1)) == 0
            s = jnp.where(mtile, 1e-8, s)
            s = s - jnp.max(s, axis=1, keepdims=True)
            p = jnp.exp(s)
            p = p / jnp.sum(p, axis=1, keepdims=True)
            ctx_full = jnp.dot(p, v, preferred_element_type=jnp.float32)
            ctx = jnp.sum((ctx_full * hmask).reshape(H, SEQ, ATT), axis=0)
            out_ref[b] = jnp.dot(ctx, Wd_ref[...],
                                 preferred_element_type=jnp.float32) \
                + bd_ref[...]


def kernel(f, route, mask, edge, node, A, W_node, emb_edges, W_ev, We_U,
           be_U, We_V, be_V, Wn_U, bn_U, Wn_V, bn_V, g_e, b_e, g_n, b_n,
           W_el, b_el, emb_time, Wq, bq, Wk, bk, Wv, bv, Wd, bd):
    f32 = jnp.float32
    ti = f[:, 0].astype(jnp.int32)
    tidx = f[:, 1].astype(jnp.int32).reshape(B, 1)
    route = route.astype(jnp.int32)
    ridx = (route[:, :, 0] * N + route[:, :, 1]).reshape(B, SEQ, 1)
    mask = mask.astype(jnp.int32)
    A3 = A.reshape(T, NN, 1)
    edge_v = edge[:, :, :, 2:].reshape(T, NN, EVD)

    def ti_map(i, ti):
        sel = jnp.where(i < B, i, jnp.where(i < 2 * B, i - B, 0))
        return (ti[sel], 0, 0)

    def b_map(i, ti):
        return (jnp.where((i >= B) & (i < 2 * B), i - B, 0), 0, 0)

    def c0(i, ti):
        return (0, 0)

    def c03(i, ti):
        return (0, 0, 0)

    w_spec = pl.BlockSpec((HID, HID), c0)
    b_spec = pl.BlockSpec((1, HID), c0)

    grid_spec = pltpu.PrefetchScalarGridSpec(
        num_scalar_prefetch=1,
        grid=(2 * B + 1,),
        in_specs=[
            pl.BlockSpec((1, NN, EVD), ti_map),
            pl.BlockSpec((1, N, NODE_DIM + 1), ti_map),
            pl.BlockSpec((1, NN, 1), ti_map),
            pl.BlockSpec((VOC, HID // 2), c0),
            pl.BlockSpec((EVD, HID // 2), c0),
            pl.BlockSpec((NODE_DIM, HID), c0),
            w_spec, b_spec, w_spec, b_spec,
            w_spec, b_spec, w_spec, b_spec,
            b_spec, b_spec, b_spec, b_spec,
            w_spec, b_spec, w_spec, b_spec,
            pl.BlockSpec((1, SEQ, 1), b_map),
            b_spec, b_spec,
            pl.BlockSpec((B, 1), c0),
            pl.BlockSpec((B, SEQ, SEQ), c03),
            pl.BlockSpec((SEQ, EOD), c0),
            pl.BlockSpec((T, TED), c0),
            pl.BlockSpec((HS, ATT), c0),
            pl.BlockSpec((HID, EOD), c0),
            pl.BlockSpec((1, EOD), c0),
            pl.BlockSpec((ATT, ATT), c0), pl.BlockSpec((1, ATT), c0),
            pl.BlockSpec((ATT, ATT), c0), pl.BlockSpec((1, ATT), c0),
            pl.BlockSpec((ATT, ATT), c0), pl.BlockSpec((1, ATT), c0),
            pl.BlockSpec((ATT, ATT), c0), pl.BlockSpec((1, ATT), c0),
        ],
        out_specs=pl.BlockSpec((B, SEQ, ATT), c03),
        scratch_shapes=[
            pltpu.VMEM((B * N, HID), f32), pltpu.VMEM((B * N, HID), f32),
            pltpu.VMEM((1, HID), f32), pltpu.VMEM((1, HID), f32),
            pltpu.VMEM((1, HID), f32), pltpu.VMEM((1, HID), f32),
            pltpu.VMEM((B * SEQ, HID), f32), pltpu.VMEM((B * SEQ, HID), f32),
            pltpu.VMEM((1, HID), f32), pltpu.VMEM((1, HID), f32),
        ],
    )
    out = pl.pallas_call(
        _body, grid_spec=grid_spec,
        out_shape=jax.ShapeDtypeStruct((B, SEQ, ATT), f32),
        compiler_params=pltpu.CompilerParams(
            dimension_semantics=("arbitrary",)))(
        ti, edge_v, node, A3, emb_edges, W_ev, W_node,
        We_U[0], be_U[0].reshape(1, HID), We_V[0], be_V[0].reshape(1, HID),
        Wn_U[0], bn_U[0].reshape(1, HID), Wn_V[0], bn_V[0].reshape(1, HID),
        g_e[0].reshape(1, HID), b_e[0].reshape(1, HID),
        g_n[0].reshape(1, HID), b_n[0].reshape(1, HID),
        We_U[1], be_U[1].reshape(1, HID), We_V[1], be_V[1].reshape(1, HID),
        ridx,
        g_e[1].reshape(1, HID), b_e[1].reshape(1, HID),
        tidx, mask, _pe_const(), emb_time, _hmask_const(),
        W_el, b_el.reshape(1, EOD),
        Wq, bq.reshape(1, ATT), Wk, bk.reshape(1, ATT),
        Wv, bv.reshape(1, ATT), Wd, bd.reshape(1, ATT))

    return jnp.concatenate([out.reshape(B, SEQ * ATT), f[:, 1:]], axis=1)


# speedup vs baseline: 4.2447x; 1.4971x over previous
import math

import jax
import jax.numpy as jnp
import numpy as np
from jax.experimental import pallas as pl
from jax.experimental.pallas import tpu as pltpu

B = 16; T = 24; N = 64; SEQ = 25
HID = 128; NODE_DIM = 8; VOC = 3; EVD = 5; TED = 16; EOD = 24; NL = 2
ATT = 64; H = 8; DH = ATT // H
NN = N * N
HS = H * SEQ
EPS = 1e-5
CNT_E = float(B * NN)
CNT_X = float(B * N)


def _pe_np():
    pe = []
    for pos in range(SEQ):
        row = []
        for ii in range(0, EOD, 2):
            row.append(math.sin(pos / 10000 ** (2 * ii / EOD)))
            row.append(math.cos(pos / 10000 ** (2 * ii / EOD)))
        pe.append(row)
    return np.array(pe, dtype=np.float32)


def _hmask_pe_const():
    m = np.zeros((HS + SEQ, ATT), dtype=np.float32)
    for h in range(H):
        m[h * SEQ:(h + 1) * SEQ, h * DH:(h + 1) * DH] = 1.0
    m[HS:, EOD:2 * EOD] = _pe_np()
    return jnp.asarray(m)


def _dot0(a, b):
    return jax.lax.dot_general(a, b, (((0,), (0,)), ((), ())),
                               preferred_element_type=jnp.float32)


def _body(tix_ref, evT_ref, nodeT_ref, Arow_ref, Wsm_ref, W_node_ref,
          Wbig_ref, Vbig_ref, ridx_ref, maskP_ref, embtP_ref, WattP_ref,
          battP_ref,
          out_ref,
          x0_s, xtmp0_s, se_s, sse_s, sx_s, ssx_s,
          re1_s, retmp1_s, se1_s, sse1_s):
    i = pl.program_id(0)

    W_ev = Wsm_ref[0:EVD, :]
    emb = Wsm_ref[EVD:EVD + VOC, :]
    WeU0 = Wbig_ref[0:HID, :]
    WeV0 = Wbig_ref[HID:2 * HID, :]
    WnU0 = Wbig_ref[2 * HID:3 * HID, :]
    WnV0 = Wbig_ref[3 * HID:4 * HID, :]
    WeU1 = Wbig_ref[4 * HID:5 * HID, :]
    WeV1 = Wbig_ref[5 * HID:6 * HID, :]
    beU0 = Vbig_ref[0:1, :]; beV0 = Vbig_ref[1:2, :]
    bnU0 = Vbig_ref[2:3, :]; bnV0 = Vbig_ref[3:4, :]
    ge0 = Vbig_ref[4:5, :]; be0 = Vbig_ref[5:6, :]
    gn0 = Vbig_ref[6:7, :]; bn0 = Vbig_ref[7:8, :]
    beU1 = Vbig_ref[8:9, :]; beV1 = Vbig_ref[9:10, :]
    ge1 = Vbig_ref[10:11, :]; be1 = Vbig_ref[11:12, :]

    @pl.when(i == 0)
    def _zero_a():
        z = jnp.zeros((1, HID), jnp.float32)
        se_s[...] = z; sse_s[...] = z; sx_s[...] = z; ssx_s[...] = z

    @pl.when(i < B)
    def _phase_a():
        evT = evT_ref[0]
        onehotT = (Arow_ref[0] == jax.lax.broadcasted_iota(
            jnp.int32, (VOC, NN), 0)).astype(jnp.float32)
        x0 = _dot0(nodeT_ref[0][1:, :], W_node_ref[...])
        x0_s[pl.ds(i * N, N), :] = x0

        W1 = jnp.dot(W_ev, WeU0[:HID // 2, :],
                     preferred_element_type=jnp.float32)
        W2 = jnp.dot(emb, WeU0[HID // 2:, :],
                     preferred_element_type=jnp.float32)
        Ue = _dot0(evT, W1) + _dot0(onehotT, W2)
        Vx = jnp.dot(x0, WeV0, preferred_element_type=jnp.float32) \
            + beV0 + beU0
        Vxi = Vx - beU0
        e_tmp = Ue.reshape(N, N, HID) + Vx[None, :, :] + Vxi[:, None, :]
        e_tmp_f = e_tmp.reshape(NN, HID)
        se_s[...] += jnp.sum(e_tmp_f, axis=0, keepdims=True)
        sse_s[...] += jnp.sum(e_tmp_f * e_tmp_f, axis=0, keepdims=True)

        gate = jax.nn.sigmoid(e_tmp)
        Vx2 = jnp.dot(x0, WnV0, preferred_element_type=jnp.float32) + bnV0
        num = jnp.sum(gate * Vx2[None, :, :], axis=1)
        den = jnp.sum(gate, axis=1)
        Ux = jnp.dot(x0, WnU0, preferred_element_type=jnp.float32) + bnU0
        x_tmp = Ux + num / (1e-20 + den)
        xtmp0_s[pl.ds(i * N, N), :] = x_tmp
        sx_s[...] += jnp.sum(x_tmp, axis=0, keepdims=True)
        ssx_s[...] += jnp.sum(x_tmp * x_tmp, axis=0, keepdims=True)

    @pl.when(i == B)
    def _zero_b():
        z = jnp.zeros((1, HID), jnp.float32)
        se1_s[...] = z; sse1_s[...] = z

    @pl.when((i >= B) & (i < 2 * B))
    def _phase_b():
        b = i - B
        m_e = se_s[...] / CNT_E
        v_e = sse_s[...] / CNT_E - m_e * m_e
        m_x = sx_s[...] / CNT_X
        v_x = ssx_s[...] / CNT_X - m_x * m_x
        sc_e = ge0[...] * jax.lax.rsqrt(v_e + EPS)
        sh_e = be0[...] - m_e * sc_e
        sc_x = gn0[...] * jax.lax.rsqrt(v_x + EPS)
        sh_x = bn0[...] - m_x * sc_x

        x0 = x0_s[pl.ds(b * N, N), :]
        x1 = x0 + jnp.maximum(xtmp0_s[pl.ds(b * N, N), :] * sc_x + sh_x,
                              0.0)

        evT = evT_ref[0]
        onehotT = (Arow_ref[0] == jax.lax.broadcasted_iota(
            jnp.int32, (VOC, NN), 0)).astype(jnp.float32)
        e0 = jnp.concatenate([_dot0(evT, W_ev), _dot0(onehotT, emb)],
                             axis=1)
        Ue0 = jnp.dot(e0, WeU0, preferred_element_type=jnp.float32)
        Vx0 = jnp.dot(x0, WeV0, preferred_element_type=jnp.float32) \
            + beV0 + beU0
        Vx0i = Vx0 - beU0
        e_tmp0 = Ue0.reshape(N, N, HID) + Vx0[None, :, :] + Vx0i[:, None, :]
        e1 = e0 + jnp.maximum(e_tmp0.reshape(NN, HID) * sc_e + sh_e, 0.0)

        Ue1 = jnp.dot(e1, WeU1, preferred_element_type=jnp.float32)
        Vx1 = jnp.dot(x1, WeV1, preferred_element_type=jnp.float32) \
            + beV1 + beU1
        Vx1i = Vx1 - beU1
        e_tmp1 = Ue1.reshape(N, N, HID) + Vx1[None, :, :] + Vx1i[:, None, :]
        e_tmp1_f = e_tmp1.reshape(NN, HID)
        se1_s[...] += jnp.sum(e_tmp1_f, axis=0, keepdims=True)
        sse1_s[...] += jnp.sum(e_tmp1_f * e_tmp1_f, axis=0, keepdims=True)

        onehotR = (ridx_ref[0] == jax.lax.broadcasted_iota(
            jnp.int32, (NN, SEQ), 0)).astype(jnp.float32)
        re1_s[pl.ds(b * SEQ, SEQ), :] = _dot0(onehotR, e1)
        retmp1_s[pl.ds(b * SEQ, SEQ), :] = _dot0(onehotR, e_tmp1_f)

    @pl.when(i == 2 * B)
    def _phase_c():
        m_e = se1_s[...] / CNT_E
        v_e = sse1_s[...] / CNT_E - m_e * m_e
        sc_e = ge1[...] * jax.lax.rsqrt(v_e + EPS)
        sh_e = be1[...] - m_e * sc_e

        WelP = WattP_ref[0:HID, :]
        Wq = WattP_ref[HID:HID + ATT, :]
        Wk = WattP_ref[HID + ATT:HID + 2 * ATT, :]
        Wv = WattP_ref[HID + 2 * ATT:HID + 3 * ATT, :]
        Wd = WattP_ref[HID + 3 * ATT:HID + 4 * ATT, :]
        bq = battP_ref[0:1, :]; bk = battP_ref[1:2, :]
        bv = battP_ref[2:3, :]; bd = battP_ref[3:4, :]
        belP = battP_ref[4:5, :]
        hmask = embtP_ref[0:HS, :]
        peS = embtP_ref[HS:HS + SEQ, :]

        base = jnp.broadcast_to(belP, (SEQ, ATT)) + peS
        scale = 1.0 / math.sqrt(DH)

        for b in range(B):
            rows = re1_s[b * SEQ:(b + 1) * SEQ, :] + jnp.maximum(
                retmp1_s[b * SEQ:(b + 1) * SEQ, :] * sc_e + sh_e, 0.0)
            t = tix_ref[1, b]
            t_row = embtP_ref[pl.ds(HS + SEQ + t, 1), :]
            R = jnp.dot(rows, WelP, preferred_element_type=jnp.float32) \
                + base + jnp.broadcast_to(t_row, (SEQ, ATT))

            q = jnp.dot(R, Wq, preferred_element_type=jnp.float32) + bq
            k = jnp.dot(R, Wk, preferred_element_type=jnp.float32) + bk
            v = jnp.dot(R, Wv, preferred_element_type=jnp.float32) + bv
            q_blk = jnp.tile(q, (H, 1)) * hmask
            s = jax.lax.dot_general(
                q_blk, k, (((1,), (1,)), ((), ())),
                preferred_element_type=jnp.float32) * scale
            mrow = maskP_ref[b:b + 1, :]
            m25 = jnp.concatenate(
                [mrow[:, ss * SEQ:(ss + 1) * SEQ] for ss in range(SEQ)],
                axis=0)
            mtile = jnp.tile(m25, (H, 1)) == 0
            s = jnp.where(mtile, 1e-8, s)
            s = s - jnp.max(s, axis=1, keepdims=True)
            p = jnp.exp(s)
            p = p / jnp.sum(p, axis=1, keepdims=True)
            ctx_full = jnp.dot(p, v, preferred_element_type=jnp.float32)
            ctx = jnp.sum((ctx_full * hmask).reshape(H, SEQ, ATT), axis=0)
            out_ref[pl.ds(b * SEQ, SEQ), :] = jnp.dot(
                ctx, Wd, preferred_element_type=jnp.float32) + bd


def kernel(f, route, mask, edge, node, A, W_node, emb_edges, W_ev, We_U,
           be_U, We_V, be_V, Wn_U, bn_U, Wn_V, bn_V, g_e, b_e, g_n, b_n,
           W_el, b_el, emb_time, Wq, bq, Wk, bk, Wv, bv, Wd, bd):
    f32 = jnp.float32
    ti = f[:, 0].astype(jnp.int32)
    tidx = f[:, 1].astype(jnp.int32)
    tix = jnp.stack([ti, tidx], axis=0)
    route = route.astype(jnp.int32)
    ridx = (route[:, :, 0] * N + route[:, :, 1]).reshape(B, 1, SEQ)
    maskP = mask.astype(jnp.int32).reshape(B, SEQ * SEQ)
    Arow = A.reshape(T, 1, NN)
    evT = edge[:, :, :, 2:].transpose(0, 3, 1, 2).reshape(T, EVD, NN)
    nodeT = node.transpose(0, 2, 1)

    Wsm = jnp.concatenate([W_ev, emb_edges], axis=0)
    Wbig = jnp.concatenate(
        [We_U[0], We_V[0], Wn_U[0], Wn_V[0], We_U[1], We_V[1]], axis=0)
    Vbig = jnp.stack(
        [be_U[0], be_V[0], bn_U[0], bn_V[0], g_e[0], b_e[0], g_n[0],
         b_n[0], be_U[1], be_V[1], g_e[1], b_e[1]], axis=0)
    WelP = jnp.concatenate(
        [W_el, jnp.zeros((HID, ATT - EOD), f32)], axis=1)
    WattP = jnp.concatenate([WelP, Wq, Wk, Wv, Wd], axis=0)
    belP = jnp.concatenate([b_el, jnp.zeros((ATT - EOD,), f32)])
    battP = jnp.stack([bq, bk, bv, bd, belP], axis=0)
    embtS = jnp.concatenate(
        [jnp.zeros((T, 2 * EOD), f32), emb_time], axis=1)
    embtP = jnp.concatenate([_hmask_pe_const(), embtS], axis=0)

    def ti_map(i, tix):
        sel = jnp.where(i < B, i, jnp.where(i < 2 * B, i - B, 0))
        return (tix[0, sel], 0, 0)

    def b_map(i, tix):
        return (jnp.where((i >= B) & (i < 2 * B), i - B, 0), 0, 0)

    def c02(i, tix):
        return (0, 0)

    grid_spec = pltpu.PrefetchScalarGridSpec(
        num_scalar_prefetch=1,
        grid=(2 * B + 1,),
        in_specs=[
            pl.BlockSpec((1, EVD, NN), ti_map),
            pl.BlockSpec((1, NODE_DIM + 1, N), ti_map),
            pl.BlockSpec((1, 1, NN), ti_map),
            pl.BlockSpec((EVD + VOC, HID // 2), c02),
            pl.BlockSpec((NODE_DIM, HID), c02),
            pl.BlockSpec((6 * HID, HID), c02),
            pl.BlockSpec((12, HID), c02),
            pl.BlockSpec((1, 1, SEQ), b_map),
            pl.BlockSpec((B, SEQ * SEQ), c02),
            pl.BlockSpec((HS + SEQ + T, ATT), c02),
            pl.BlockSpec((HID + 4 * ATT, ATT), c02),
            pl.BlockSpec((5, ATT), c02),
        ],
        out_specs=pl.BlockSpec((B * SEQ, ATT), c02),
        scratch_shapes=[
            pltpu.VMEM((B * N, HID), f32), pltpu.VMEM((B * N, HID), f32),
            pltpu.VMEM((1, HID), f32), pltpu.VMEM((1, HID), f32),
            pltpu.VMEM((1, HID), f32), pltpu.VMEM((1, HID), f32),
            pltpu.VMEM((B * SEQ, HID), f32), pltpu.VMEM((B * SEQ, HID), f32),
            pltpu.VMEM((1, HID), f32), pltpu.VMEM((1, HID), f32),
        ],
    )
    out = pl.pallas_call(
        _body, grid_spec=grid_spec,
        out_shape=jax.ShapeDtypeStruct((B * SEQ, ATT), f32),
        compiler_params=pltpu.CompilerParams(
            dimension_semantics=("arbitrary",)))(
        tix, evT, nodeT, Arow, Wsm, W_node, Wbig, Vbig, ridx, maskP,
        embtP, WattP, battP)

    return jnp.concatenate([out.reshape(B, SEQ * ATT), f[:, 1:]], axis=1)


# speedup vs baseline: 4.3927x; 1.0349x over previous
import math

import jax
import jax.numpy as jnp
import numpy as np
from jax.experimental import pallas as pl
from jax.experimental.pallas import tpu as pltpu

B = 16; T = 24; N = 64; SEQ = 25
HID = 128; NODE_DIM = 8; VOC = 3; EVD = 5; TED = 16; EOD = 24; NL = 2
ATT = 64; H = 8; DH = ATT // H
NN = N * N
HS = H * SEQ
EPS = 1e-5
CNT_E = float(B * NN)
CNT_X = float(B * N)


def _pe_np():
    pe = []
    for pos in range(SEQ):
        row = []
        for ii in range(0, EOD, 2):
            row.append(math.sin(pos / 10000 ** (2 * ii / EOD)))
            row.append(math.cos(pos / 10000 ** (2 * ii / EOD)))
        pe.append(row)
    return np.array(pe, dtype=np.float32)


def _hmask_pe_const():
    m = np.zeros((HS + SEQ, ATT), dtype=np.float32)
    for h in range(H):
        m[h * SEQ:(h + 1) * SEQ, h * DH:(h + 1) * DH] = 1.0
    m[HS:, EOD:2 * EOD] = _pe_np()
    return jnp.asarray(m)


def _stile_const():
    m = np.zeros((HS, ATT), dtype=np.float32)
    for h in range(H):
        m[h * SEQ:(h + 1) * SEQ, 0:SEQ] = np.eye(SEQ, dtype=np.float32)
    return jnp.asarray(m)


def _dot0(a, b):
    return jax.lax.dot_general(a, b, (((0,), (0,)), ((), ())),
                               preferred_element_type=jnp.float32)


def _body(tix_ref, evT_ref, nodeT_ref, Arow_ref, Wsm_ref, W_node_ref,
          Wbig_ref, Vbig_ref, ridx_ref, maskP_ref, embtP_ref, WattP_ref,
          battP_ref,
          out_ref,
          x0_s, xtmp0_s, se_s, sse_s, sx_s, ssx_s,
          re1_s, retmp1_s, se1_s, sse1_s, tqkv_s, ctx_s):
    i = pl.program_id(0)

    W_ev = Wsm_ref[0:EVD, :]
    emb = Wsm_ref[EVD:EVD + VOC, :]
    WeU0 = Wbig_ref[0:HID, :]
    WeV0 = Wbig_ref[HID:2 * HID, :]
    WnU0 = Wbig_ref[2 * HID:3 * HID, :]
    WnV0 = Wbig_ref[3 * HID:4 * HID, :]
    WeU1 = Wbig_ref[4 * HID:5 * HID, :]
    WeV1 = Wbig_ref[5 * HID:6 * HID, :]
    beU0 = Vbig_ref[0:1, :]; beV0 = Vbig_ref[1:2, :]
    bnU0 = Vbig_ref[2:3, :]; bnV0 = Vbig_ref[3:4, :]
    ge0 = Vbig_ref[4:5, :]; be0 = Vbig_ref[5:6, :]
    gn0 = Vbig_ref[6:7, :]; bn0 = Vbig_ref[7:8, :]
    beU1 = Vbig_ref[8:9, :]; beV1 = Vbig_ref[9:10, :]
    ge1 = Vbig_ref[10:11, :]; be1 = Vbig_ref[11:12, :]

    @pl.when(i == 0)
    def _zero_a():
        z = jnp.zeros((1, HID), jnp.float32)
        se_s[...] = z; sse_s[...] = z; sx_s[...] = z; ssx_s[...] = z

    @pl.when(i < B)
    def _phase_a():
        evT = evT_ref[0]
        onehotT = (Arow_ref[0] == jax.lax.broadcasted_iota(
            jnp.int32, (VOC, NN), 0)).astype(jnp.float32)
        x0 = _dot0(nodeT_ref[0][1:, :], W_node_ref[...])
        x0_s[pl.ds(i * N, N), :] = x0

        W1 = jnp.dot(W_ev, WeU0[:HID // 2, :],
                     preferred_element_type=jnp.float32)
        W2 = jnp.dot(emb, WeU0[HID // 2:, :],
                     preferred_element_type=jnp.float32)
        Ue = _dot0(evT, W1) + _dot0(onehotT, W2)
        Vx = jnp.dot(x0, WeV0, preferred_element_type=jnp.float32) \
            + beV0 + beU0
        Vxi = Vx - beU0
        e_tmp = Ue.reshape(N, N, HID) + Vx[None, :, :] + Vxi[:, None, :]
        e_tmp_f = e_tmp.reshape(NN, HID)
        se_s[...] += jnp.sum(e_tmp_f, axis=0, keepdims=True)
        sse_s[...] += jnp.sum(e_tmp_f * e_tmp_f, axis=0, keepdims=True)

        gate = jax.nn.sigmoid(e_tmp)
        Vx2 = jnp.dot(x0, WnV0, preferred_element_type=jnp.float32) + bnV0
        num = jnp.sum(gate * Vx2[None, :, :], axis=1)
        den = jnp.sum(gate, axis=1)
        Ux = jnp.dot(x0, WnU0, preferred_element_type=jnp.float32) + bnU0
        x_tmp = Ux + num / (1e-20 + den)
        xtmp0_s[pl.ds(i * N, N), :] = x_tmp
        sx_s[...] += jnp.sum(x_tmp, axis=0, keepdims=True)
        ssx_s[...] += jnp.sum(x_tmp * x_tmp, axis=0, keepdims=True)

    @pl.when(i == B)
    def _zero_b():
        z = jnp.zeros((1, HID), jnp.float32)
        se1_s[...] = z; sse1_s[...] = z

    @pl.when((i >= B) & (i < 2 * B))
    def _phase_b():
        b = i - B
        m_e = se_s[...] / CNT_E
        v_e = sse_s[...] / CNT_E - m_e * m_e
        m_x = sx_s[...] / CNT_X
        v_x = ssx_s[...] / CNT_X - m_x * m_x
        sc_e = ge0[...] * jax.lax.rsqrt(v_e + EPS)
        sh_e = be0[...] - m_e * sc_e
        sc_x = gn0[...] * jax.lax.rsqrt(v_x + EPS)
        sh_x = bn0[...] - m_x * sc_x

        x0 = x0_s[pl.ds(b * N, N), :]
        x1 = x0 + jnp.maximum(xtmp0_s[pl.ds(b * N, N), :] * sc_x + sh_x,
                              0.0)

        evT = evT_ref[0]
        onehotT = (Arow_ref[0] == jax.lax.broadcasted_iota(
            jnp.int32, (VOC, NN), 0)).astype(jnp.float32)
        e0 = jnp.concatenate([_dot0(evT, W_ev), _dot0(onehotT, emb)],
                             axis=1)
        Ue0 = jnp.dot(e0, WeU0, preferred_element_type=jnp.float32)
        Vx0 = jnp.dot(x0, WeV0, preferred_element_type=jnp.float32) \
            + beV0 + beU0
        Vx0i = Vx0 - beU0
        e_tmp0 = Ue0.reshape(N, N, HID) + Vx0[None, :, :] + Vx0i[:, None, :]
        e1 = e0 + jnp.maximum(e_tmp0.reshape(NN, HID) * sc_e + sh_e, 0.0)

        Ue1 = jnp.dot(e1, WeU1, preferred_element_type=jnp.float32)
        Vx1 = jnp.dot(x1, WeV1, preferred_element_type=jnp.float32) \
            + beV1 + beU1
        Vx1i = Vx1 - beU1
        e_tmp1 = Ue1.reshape(N, N, HID) + Vx1[None, :, :] + Vx1i[:, None, :]
        e_tmp1_f = e_tmp1.reshape(NN, HID)
        se1_s[...] += jnp.sum(e_tmp1_f, axis=0, keepdims=True)
        sse1_s[...] += jnp.sum(e_tmp1_f * e_tmp1_f, axis=0, keepdims=True)

        onehotR = (ridx_ref[0] == jax.lax.broadcasted_iota(
            jnp.int32, (NN, SEQ), 0)).astype(jnp.float32)
        re1_s[pl.ds(b * SEQ, SEQ), :] = _dot0(onehotR, e1)
        retmp1_s[pl.ds(b * SEQ, SEQ), :] = _dot0(onehotR, e_tmp1_f)

    @pl.when(i == 2 * B)
    def _phase_c():
        m_e = se1_s[...] / CNT_E
        v_e = sse1_s[...] / CNT_E - m_e * m_e
        sc_e = ge1[...] * jax.lax.rsqrt(v_e + EPS)
        sh_e = be1[...] - m_e * sc_e

        WelP = WattP_ref[0:HID, :]
        Wq = WattP_ref[HID:HID + ATT, :]
        Wk = WattP_ref[HID + ATT:HID + 2 * ATT, :]
        Wv = WattP_ref[HID + 2 * ATT:HID + 3 * ATT, :]
        Wd = WattP_ref[HID + 3 * ATT:HID + 4 * ATT, :]
        bq = battP_ref[0:1, :]; bk = battP_ref[1:2, :]
        bv = battP_ref[2:3, :]; bd = battP_ref[3:4, :]
        belP = battP_ref[4:5, :]
        hmask = embtP_ref[0:HS, :]
        peS = embtP_ref[HS:HS + SEQ, :]
        embt = embtP_ref[HS + SEQ:HS + SEQ + T, :]
        stile = embtP_ref[HS + SEQ + T:, :][:, 0:SEQ]

        Wqkv = jnp.concatenate([Wq, Wk, Wv], axis=1)
        bqkv = jnp.concatenate([bq, bk, bv], axis=1)
        base = jnp.broadcast_to(belP, (SEQ, ATT)) + peS
        baseQKV = jnp.dot(base, Wqkv,
                          preferred_element_type=jnp.float32) + bqkv
        tqkv_s[...] = jnp.dot(embt, Wqkv,
                              preferred_element_type=jnp.float32)
        rows_all = re1_s[...] + jnp.maximum(
            retmp1_s[...] * sc_e + sh_e, 0.0)
        R0 = jnp.dot(rows_all, WelP, preferred_element_type=jnp.float32)
        qkv_all = jnp.dot(R0, Wqkv, preferred_element_type=jnp.float32)

        scale = 1.0 / math.sqrt(DH)
        for b in range(B):
            t = tix_ref[1, b]
            trow = tqkv_s[pl.ds(t, 1), :]
            qkv = qkv_all[b * SEQ:(b + 1) * SEQ, :] + baseQKV \
                + jnp.broadcast_to(trow, (SEQ, 3 * ATT))
            q = qkv[:, 0:ATT]; k = qkv[:, ATT:2 * ATT]; v = qkv[:, 2 * ATT:]
            q_blk = jnp.dot(stile, q,
                            preferred_element_type=jnp.float32) * hmask
            s = jax.lax.dot_general(
                q_blk, k, (((1,), (1,)), ((), ())),
                preferred_element_type=jnp.float32) * scale
            mtile = maskP_ref[b * HS:(b + 1) * HS, :]
            s = jnp.where(mtile == 0.0, 1e-8, s)
            s = s - jnp.max(s, axis=1, keepdims=True)
            p = jnp.exp(s)
            p = p / jnp.sum(p, axis=1, keepdims=True)
            ctx_full = jnp.dot(p, v, preferred_element_type=jnp.float32)
            ctx_s[pl.ds(b * SEQ, SEQ), :] = _dot0(stile, ctx_full * hmask)
        out_ref[...] = jnp.dot(ctx_s[...], Wd,
                               preferred_element_type=jnp.float32) + bd


def kernel(f, route, mask, edge, node, A, W_node, emb_edges, W_ev, We_U,
           be_U, We_V, be_V, Wn_U, bn_U, Wn_V, bn_V, g_e, b_e, g_n, b_n,
           W_el, b_el, emb_time, Wq, bq, Wk, bk, Wv, bv, Wd, bd):
    f32 = jnp.float32
    ti = f[:, 0].astype(jnp.int32)
    tidx = f[:, 1].astype(jnp.int32)
    tix = jnp.stack([ti, tidx], axis=0)
    route = route.astype(jnp.int32)
    ridx = (route[:, :, 0] * N + route[:, :, 1]).reshape(B, 1, SEQ)
    maskP = jnp.broadcast_to(
        mask.astype(f32).reshape(B, 1, SEQ, SEQ),
        (B, H, SEQ, SEQ)).reshape(B * HS, SEQ)
    Arow = A.reshape(T, 1, NN)
    evT = edge[:, :, :, 2:].transpose(0, 3, 1, 2).reshape(T, EVD, NN)
    nodeT = node.transpose(0, 2, 1)

    Wsm = jnp.concatenate([W_ev, emb_edges], axis=0)
    Wbig = jnp.concatenate(
        [We_U[0], We_V[0], Wn_U[0], Wn_V[0], We_U[1], We_V[1]], axis=0)
    Vbig = jnp.stack(
        [be_U[0], be_V[0], bn_U[0], bn_V[0], g_e[0], b_e[0], g_n[0],
         b_n[0], be_U[1], be_V[1], g_e[1], b_e[1]], axis=0)
    WelP = jnp.concatenate(
        [W_el, jnp.zeros((HID, ATT - EOD), f32)], axis=1)
    WattP = jnp.concatenate([WelP, Wq, Wk, Wv, Wd], axis=0)
    belP = jnp.concatenate([b_el, jnp.zeros((ATT - EOD,), f32)])
    battP = jnp.stack([bq, bk, bv, bd, belP], axis=0)
    embtS = jnp.concatenate(
        [jnp.zeros((T, 2 * EOD), f32), emb_time], axis=1)
    embtP = jnp.concatenate([_hmask_pe_const(), embtS, _stile_const()],
                            axis=0)

    def ti_map(i, tix):
        sel = jnp.where(i < B, i, jnp.where(i < 2 * B, i - B, 0))
        return (tix[0, sel], 0, 0)

    def b_map(i, tix):
        return (jnp.where((i >= B) & (i < 2 * B), i - B, 0), 0, 0)

    def c02(i, tix):
        return (0, 0)

    grid_spec = pltpu.PrefetchScalarGridSpec(
        num_scalar_prefetch=1,
        grid=(2 * B + 1,),
        in_specs=[
            pl.BlockSpec((1, EVD, NN), ti_map),
            pl.BlockSpec((1, NODE_DIM + 1, N), ti_map),
            pl.BlockSpec((1, 1, NN), ti_map),
            pl.BlockSpec((EVD + VOC, HID // 2), c02),
            pl.BlockSpec((NODE_DIM, HID), c02),
            pl.BlockSpec((6 * HID, HID), c02),
            pl.BlockSpec((12, HID), c02),
            pl.BlockSpec((1, 1, SEQ), b_map),
            pl.BlockSpec((B * HS, SEQ), c02),
            pl.BlockSpec((HS + SEQ + T + HS, ATT), c02),
            pl.BlockSpec((HID + 4 * ATT, ATT), c02),
            pl.BlockSpec((5, ATT), c02),
        ],
        out_specs=pl.BlockSpec((B * SEQ, ATT), c02),
        scratch_shapes=[
            pltpu.VMEM((B * N, HID), f32), pltpu.VMEM((B * N, HID), f32),
            pltpu.VMEM((1, HID), f32), pltpu.VMEM((1, HID), f32),
            pltpu.VMEM((1, HID), f32), pltpu.VMEM((1, HID), f32),
            pltpu.VMEM((B * SEQ, HID), f32), pltpu.VMEM((B * SEQ, HID), f32),
            pltpu.VMEM((1, HID), f32), pltpu.VMEM((1, HID), f32),
            pltpu.VMEM((T, 3 * ATT), f32), pltpu.VMEM((B * SEQ, ATT), f32),
        ],
    )
    out = pl.pallas_call(
        _body, grid_spec=grid_spec,
        out_shape=jax.ShapeDtypeStruct((B * SEQ, ATT), f32),
        compiler_params=pltpu.CompilerParams(
            dimension_semantics=("arbitrary",)))(
        tix, evT, nodeT, Arow, Wsm, W_node, Wbig, Vbig, ridx, maskP,
        embtP, WattP, battP)

    return jnp.concatenate([out.reshape(B, SEQ * ATT), f[:, 1:]], axis=1)
